# Initial kernel scaffold; baseline (speedup 1.0000x reference)
import jax
import jax.numpy as jnp
from jax.experimental import pallas as pl


def kernel(edge_index, edge_type, user_indices, item_indices, entity_table, relation_table, W_gat, att_src, att_dst, bias_gat, W_pred, b_pred):
    raise NotImplementedError("write your pallas kernel here")


# speedup vs baseline: 12.4928x; 12.4928x over previous
import functools

import jax
import jax.numpy as jnp
from jax import lax
from jax.experimental import pallas as pl
from jax.experimental.pallas import tpu as pltpu
from jax.experimental.pallas import tpu_sc as plsc

N = 10000
D = 128
H = 8
E = 320000
B = 4096
DP = 144
PAD = DP - D

NC = 2
NS = 16
L = 16

RB = 1000
NB = N // RB

HPC = H // NC
EPT = E // NS
C = 80
NCHUNK = EPT // C
NA = 10240
RPT = NA // NS
ZR = 32

PPB = B // (NC * NS)


def _stage1_body(x_ref, w_ref, asrc_ref, adst_ref, xp_ref, s_ref, t_ref):
    i = pl.program_id(0)
    h = pl.program_id(1)
    xp = jnp.dot(x_ref[...], w_ref[...], preferred_element_type=jnp.float32)
    xp_ref[:, :D] = xp
    pad = (lax.broadcasted_iota(jnp.int32, (RB, PAD), 1) == 0).astype(jnp.float32)
    xp_ref[:, D:] = pad
    asrc = asrc_ref[pl.ds(h, 1), :]
    adst = adst_ref[pl.ds(h, 1), :]
    s_ref[...] = jnp.sum(xp * asrc, axis=1).reshape(1, 1, 1, RB)
    t_ref[...] = jnp.sum(xp * adst, axis=1).reshape(1, 1, 1, RB)


_stage1 = pl.pallas_call(
    _stage1_body,
    grid=(NB, H),
    in_specs=[
        pl.BlockSpec((RB, D), lambda i, h: (i, 0)),
        pl.BlockSpec((D, D), lambda i, h: (0, h)),
        pl.BlockSpec((H, D), lambda i, h: (0, 0)),
        pl.BlockSpec((H, D), lambda i, h: (0, 0)),
    ],
    out_specs=[
        pl.BlockSpec((RB, DP), lambda i, h: (h * NB + i, 0)),
        pl.BlockSpec((1, 1, 1, RB), lambda i, h: (h, i, 0, 0)),
        pl.BlockSpec((1, 1, 1, RB), lambda i, h: (h, i, 0, 0)),
    ],
    out_shape=[
        jax.ShapeDtypeStruct((H * N, DP), jnp.float32),
        jax.ShapeDtypeStruct((H, NB, 1, RB), jnp.float32),
        jax.ShapeDtypeStruct((H, NB, 1, RB), jnp.float32),
    ],
)


def _stage2_body(xp_hbm, st_hbm, tt_hbm, src_hbm, dst_hbm, acc_hbm,
                 s_v, t_v, rows_v, srcv, dstv, idxv, wv, zbuf, acc_sh, sem):
    cid = lax.axis_index("c")
    sid = lax.axis_index("s")

    def zrow(i, carry):
        for j in range(DP // L):
            zbuf[i, pl.ds(L * j, L)] = jnp.zeros((L,), jnp.float32)
        return carry
    lax.fori_loop(0, ZR, zrow, 0)

    ebase = sid * EPT
    for hh in range(HPC):
        h = cid * HPC + hh
        hbase = h * N
        abase = h * NA
        for k in range(RPT // ZR):
            pltpu.sync_copy(zbuf, acc_sh.at[pl.ds(sid * RPT + k * ZR, ZR)])
        pltpu.sync_copy(st_hbm.at[h], s_v)
        pltpu.sync_copy(tt_hbm.at[h], t_v)
        plsc.subcore_barrier()

        def chunk(ci, carry):
            eb = ebase + ci * C
            pltpu.sync_copy(src_hbm.at[pl.ds(eb, C)], srcv)
            pltpu.sync_copy(dst_hbm.at[pl.ds(eb, C)], dstv)
            for j in range(C // L):
                s16 = srcv[pl.ds(L * j, L)]
                d16 = dstv[pl.ds(L * j, L)]
                idxv[pl.ds(L * j, L)] = s16 + hbase
                e16 = (plsc.load_gather(s_v, [s16])
                       + plsc.load_gather(t_v, [d16]))
                e16 = jnp.where(e16 >= 0, e16, 0.2 * e16)
                wv[pl.ds(L * j, L)] = jnp.exp(e16)
            pltpu.async_copy(xp_hbm.at[idxv], rows_v, sem).wait()

            def scale(gi, c2):
                w16 = wv[pl.ds(L * gi, L)]
                for e in range(L):
                    w = w16[e]
                    row = L * gi + e
                    for j in range(DP // L):
                        rows_v[row, pl.ds(L * j, L)] = (
                            rows_v[row, pl.ds(L * j, L)] * w)
                return c2
            lax.fori_loop(0, C // L, scale, 0)
            pltpu.sync_copy(rows_v, acc_sh.at[dstv], add=True)
            return carry
        lax.fori_loop(0, NCHUNK, chunk, 0)
        plsc.subcore_barrier()
        pltpu.sync_copy(acc_sh.at[pl.ds(sid * RPT, RPT)],
                        acc_hbm.at[pl.ds(abase + sid * RPT, RPT)])
        plsc.subcore_barrier()


_stage2 = functools.partial(
    pl.kernel,
    out_type=jax.ShapeDtypeStruct((H * NA, DP), jnp.float32),
    mesh=plsc.VectorSubcoreMesh(core_axis_name="c", subcore_axis_name="s",
                                num_cores=NC, num_subcores=NS),
    compiler_params=pltpu.CompilerParams(needs_layout_passes=False, use_tc_tiling_on_sc=False),
    scratch_types=[
        pltpu.VMEM((N,), jnp.float32),
        pltpu.VMEM((N,), jnp.float32),
        pltpu.VMEM((C, DP), jnp.float32),
        pltpu.VMEM((C,), jnp.int32),
        pltpu.VMEM((C,), jnp.int32),
        pltpu.VMEM((C,), jnp.int32),
        pltpu.VMEM((C,), jnp.float32),
        pltpu.VMEM((ZR, DP), jnp.float32),
        pltpu.VMEM_SHARED((NA, DP), jnp.float32),
        pltpu.SemaphoreType.DMA,
    ],
)(_stage2_body)


def _stage3_body(acc_hbm, bias_hbm, uidx_hbm, iidx_hbm, wp_hbm, bv_hbm,
                 out_hbm, uidx_v, iidx_v, wp_v, bias_v, bv_v, idxA, idxB,
                 rowsA, rowsB, scores_v, sem):
    cid = lax.axis_index("c")
    sid = lax.axis_index("s")
    wid = cid * NS + sid
    pbase = wid * PPB
    pltpu.sync_copy(uidx_hbm.at[pl.ds(pbase, PPB)], uidx_v)
    pltpu.sync_copy(iidx_hbm.at[pl.ds(pbase, PPB)], iidx_v)
    pltpu.sync_copy(wp_hbm, wp_v)
    pltpu.sync_copy(bias_hbm, bias_v)
    pltpu.sync_copy(bv_hbm, bv_v)

    acc1 = jnp.zeros((L,), jnp.float32)
    acc2 = jnp.zeros((L,), jnp.float32)
    for j in range(D // L):
        bseg = bias_v[pl.ds(L * j, L)]
        acc1 = acc1 + bseg * wp_v[pl.ds(L * j, L)]
        acc2 = acc2 + bseg * wp_v[pl.ds(D + L * j, L)]
    bconst = jnp.sum(acc1) + jnp.sum(acc2) + bv_v[...][0]

    iota16 = lax.iota(jnp.int32, L)

    def group(g, carry):
        u16 = uidx_v[pl.ds(L * g, L)]
        i16 = iidx_v[pl.ds(L * g, L)]
        for r in range(H):
            idxA[pl.ds(L * r, L)] = u16 + r * NA
            idxB[pl.ds(L * r, L)] = i16 + r * NA
        pltpu.async_copy(acc_hbm.at[idxA], rowsA, sem).wait()
        pltpu.async_copy(acc_hbm.at[idxB], rowsB, sem).wait()
        sc16 = jnp.zeros((L,), jnp.float32)
        for p in range(L):
            def hrow(r, carry):
                g16, dn16 = carry
                row = L * r + p
                va = jnp.zeros((L,), jnp.float32)
                vb = jnp.zeros((L,), jnp.float32)
                for j in range(D // L):
                    va = va + rowsA[row, pl.ds(L * j, L)] * wp_v[pl.ds(L * j, L)]
                    vb = vb + rowsB[row, pl.ds(L * j, L)] * wp_v[pl.ds(D + L * j, L)]
                dna = rowsA[row, pl.ds(D, L)][0]
                dnb = rowsB[row, pl.ds(D, L)][0]
                g16 = jnp.where(iota16 == r, jnp.sum(va), g16)
                g16 = jnp.where(iota16 == r + H, jnp.sum(vb), g16)
                dn16 = jnp.where(iota16 == r, dna, dn16)
                dn16 = jnp.where(iota16 == r + H, dnb, dn16)
                return (g16, dn16)
            g16, dn16 = lax.fori_loop(
                0, H, hrow,
                (jnp.zeros((L,), jnp.float32), jnp.ones((L,), jnp.float32)))
            total = jnp.sum(g16 / (dn16 + 1e-16))
            sc16 = jnp.where(iota16 == p, total, sc16)
        sc16 = 1.0 / (1.0 + jnp.exp(-(sc16 * (1.0 / H) + bconst)))
        scores_v[pl.ds(L * g, L)] = sc16
        return carry
    lax.fori_loop(0, PPB // L, group, 0)
    pltpu.sync_copy(scores_v, out_hbm.at[pl.ds(pbase, PPB)])


_stage3 = functools.partial(
    pl.kernel,
    out_type=jax.ShapeDtypeStruct((B,), jnp.float32),
    mesh=plsc.VectorSubcoreMesh(core_axis_name="c", subcore_axis_name="s",
                                num_cores=NC, num_subcores=NS),
    compiler_params=pltpu.CompilerParams(needs_layout_passes=False, use_tc_tiling_on_sc=False),
    scratch_types=[
        pltpu.VMEM((PPB,), jnp.int32),
        pltpu.VMEM((PPB,), jnp.int32),
        pltpu.VMEM((2 * D,), jnp.float32),
        pltpu.VMEM((D,), jnp.float32),
        pltpu.VMEM((L,), jnp.float32),
        pltpu.VMEM((H * L,), jnp.int32),
        pltpu.VMEM((H * L,), jnp.int32),
        pltpu.VMEM((H * L, DP), jnp.float32),
        pltpu.VMEM((H * L, DP), jnp.float32),
        pltpu.VMEM((PPB,), jnp.float32),
        pltpu.SemaphoreType.DMA,
    ],
)(_stage3_body)


def kernel(edge_index, edge_type, user_indices, item_indices, entity_table,
           relation_table, W_gat, att_src, att_dst, bias_gat, W_pred, b_pred):
    del edge_type, relation_table
    src = edge_index[0].astype(jnp.int32)
    dst = edge_index[1].astype(jnp.int32)
    xp2, s_t, t_t = _stage1(entity_table, W_gat, att_src, att_dst)
    acc = _stage2(xp2, s_t.reshape(H, N), t_t.reshape(H, N), src, dst)
    wp = W_pred.reshape(2 * D)
    bv = jnp.concatenate([b_pred.astype(jnp.float32),
                          jnp.zeros((L - 1,), jnp.float32)])
    return _stage3(acc, bias_gat, user_indices.astype(jnp.int32),
                   item_indices.astype(jnp.int32), wp, bv)


# speedup vs baseline: 14.1624x; 1.1336x over previous
import functools

import jax
import jax.numpy as jnp
from jax import lax
from jax.experimental import pallas as pl
from jax.experimental.pallas import tpu as pltpu
from jax.experimental.pallas import tpu_sc as plsc

N = 10000
D = 128
H = 8
E = 320000
B = 4096
DP = 144
PAD = DP - D

NC = 2
NS = 16
L = 16

RB = 1000
NB = N // RB

HPC = H // NC
EPT = E // NS
C = 80
CA = 48
CB = C - CA
NCHUNK = EPT // C
NA = 10240
RPT = NA // NS
ZR = 32

PPB = B // (NC * NS)


def _stage1_body(x_ref, w_ref, asrc_ref, adst_ref, xp_ref, s_ref, t_ref):
    i = pl.program_id(0)
    h = pl.program_id(1)
    xp = jnp.dot(x_ref[...], w_ref[...], preferred_element_type=jnp.float32)
    xp_ref[:, :D] = xp
    pad = (lax.broadcasted_iota(jnp.int32, (RB, PAD), 1) == 0).astype(jnp.float32)
    xp_ref[:, D:] = pad
    asrc = asrc_ref[pl.ds(h, 1), :]
    adst = adst_ref[pl.ds(h, 1), :]
    s_ref[...] = jnp.sum(xp * asrc, axis=1).reshape(1, 1, 1, RB)
    t_ref[...] = jnp.sum(xp * adst, axis=1).reshape(1, 1, 1, RB)


_stage1 = pl.pallas_call(
    _stage1_body,
    grid=(NB, H),
    in_specs=[
        pl.BlockSpec((RB, D), lambda i, h: (i, 0)),
        pl.BlockSpec((D, D), lambda i, h: (0, h)),
        pl.BlockSpec((H, D), lambda i, h: (0, 0)),
        pl.BlockSpec((H, D), lambda i, h: (0, 0)),
    ],
    out_specs=[
        pl.BlockSpec((RB, DP), lambda i, h: (h * NB + i, 0)),
        pl.BlockSpec((1, 1, 1, RB), lambda i, h: (h, i, 0, 0)),
        pl.BlockSpec((1, 1, 1, RB), lambda i, h: (h, i, 0, 0)),
    ],
    out_shape=[
        jax.ShapeDtypeStruct((H * N, DP), jnp.float32),
        jax.ShapeDtypeStruct((H, NB, 1, RB), jnp.float32),
        jax.ShapeDtypeStruct((H, NB, 1, RB), jnp.float32),
    ],
)


def _stage2_body(xp_hbm, st_hbm, tt_hbm, src_hbm, dst_hbm, acc_hbm,
                 s_v, t_v, rows_v,
                 srcv0, dstA0, dstB0, idxA0, idxB0, wvA0, wvB0,
                 srcv1, dstA1, dstB1, idxA1, idxB1, wvA1, wvB1,
                 acc_sh, sem):
    cid = lax.axis_index("c")
    sid = lax.axis_index("s")
    bufs0 = (srcv0, dstA0, dstB0, idxA0, idxB0, wvA0, wvB0)
    bufs1 = (srcv1, dstA1, dstB1, idxA1, idxB1, wvA1, wvB1)
    ebase = sid * EPT

    def compute_chunk(ci, bufs, hbase):
        srcv, dstA, dstB, idxA, idxB, wvA, wvB = bufs
        eb = ebase + ci * C
        pltpu.sync_copy(src_hbm.at[pl.ds(eb, C)], srcv)
        pltpu.sync_copy(dst_hbm.at[pl.ds(eb, CA)], dstA)
        pltpu.sync_copy(dst_hbm.at[pl.ds(eb + CA, CB)], dstB)
        for g in range(C // L):
            ina = g < CA // L
            dref, iref, wref = (dstA, idxA, wvA) if ina else (dstB, idxB, wvB)
            lofs = L * g if ina else L * g - CA
            s16 = srcv[pl.ds(L * g, L)]
            d16 = dref[pl.ds(lofs, L)]
            iref[pl.ds(lofs, L)] = s16 + hbase
            e16 = (plsc.load_gather(s_v, [s16])
                   + plsc.load_gather(t_v, [d16]))
            e16 = jnp.where(e16 >= 0, e16, 0.2 * e16)
            wref[pl.ds(lofs, L)] = jnp.exp(e16)

    def start_gA(bufs):
        pltpu.async_copy(xp_hbm.at[bufs[3]], rows_v.at[pl.ds(0, CA)], sem)

    def wait_gA(bufs):
        pltpu.make_async_copy(xp_hbm.at[bufs[3]],
                              rows_v.at[pl.ds(0, CA)], sem).wait()

    def start_wait_gB(bufs):
        pltpu.async_copy(xp_hbm.at[bufs[4]], rows_v.at[pl.ds(CA, CB)], sem)
        return lambda: pltpu.make_async_copy(
            xp_hbm.at[bufs[4]], rows_v.at[pl.ds(CA, CB)], sem).wait()

    def scale_half(wref, base, ngroups):
        def sg(g, c2):
            off = L * g
            w16 = wref[pl.ds(off, L)]
            for e in range(L):
                row = base + off + e
                for j in range(DP // L):
                    rows_v[row, pl.ds(L * j, L)] = (
                        rows_v[row, pl.ds(L * j, L)] * w16[e])
            return c2
        lax.fori_loop(0, ngroups, sg, 0)

    def do_chunk(cur, nxt, ci_next, hbase, prefetch):
        if prefetch:
            compute_chunk(ci_next, nxt, hbase)
        wait_gA(cur)
        wait_b = start_wait_gB(cur)
        scale_half(cur[5], 0, CA // L)
        pltpu.sync_copy(rows_v.at[pl.ds(0, CA)], acc_sh.at[cur[1]], add=True)
        wait_b()
        if prefetch:
            start_gA(nxt)
        scale_half(cur[6], CA, CB // L)
        pltpu.sync_copy(rows_v.at[pl.ds(CA, CB)], acc_sh.at[cur[2]], add=True)

    def zero_rows():
        def zr(i, c2):
            for j in range(DP // L):
                rows_v[i, pl.ds(L * j, L)] = jnp.zeros((L,), jnp.float32)
            return c2
        lax.fori_loop(0, C, zr, 0)

    def headbody(hh, carry):
        h = cid * HPC + hh
        hbase = h * N
        abase = h * NA
        zero_rows()
        for k in range(RPT // C):
            pltpu.sync_copy(rows_v, acc_sh.at[pl.ds(sid * RPT + k * C, C)])
        pltpu.sync_copy(st_hbm.at[h], s_v)
        pltpu.sync_copy(tt_hbm.at[h], t_v)
        plsc.subcore_barrier()

        compute_chunk(0, bufs0, hbase)
        start_gA(bufs0)

        def pairbody(k, c2):
            ci = 2 * k
            do_chunk(bufs0, bufs1, ci + 1, hbase, True)
            do_chunk(bufs1, bufs0, ci + 2, hbase, True)
            return c2
        lax.fori_loop(0, NCHUNK // 2, pairbody, 0)
        wait_gA(bufs0)

        plsc.subcore_barrier()
        pltpu.sync_copy(acc_sh.at[pl.ds(sid * RPT, RPT)],
                        acc_hbm.at[pl.ds(abase + sid * RPT, RPT)])
        plsc.subcore_barrier()
        return carry
    lax.fori_loop(0, HPC, headbody, 0)


_stage2 = functools.partial(
    pl.kernel,
    out_type=jax.ShapeDtypeStruct((H * NA, DP), jnp.float32),
    mesh=plsc.VectorSubcoreMesh(core_axis_name="c", subcore_axis_name="s",
                                num_cores=NC, num_subcores=NS),
    compiler_params=pltpu.CompilerParams(needs_layout_passes=False, use_tc_tiling_on_sc=False),
    scratch_types=(
        [pltpu.VMEM((N,), jnp.float32),
         pltpu.VMEM((N,), jnp.float32),
         pltpu.VMEM((C, DP), jnp.float32)]
        + 2 * [pltpu.VMEM((C,), jnp.int32),
               pltpu.VMEM((CA,), jnp.int32),
               pltpu.VMEM((CB,), jnp.int32),
               pltpu.VMEM((CA,), jnp.int32),
               pltpu.VMEM((CB,), jnp.int32),
               pltpu.VMEM((CA,), jnp.float32),
               pltpu.VMEM((CB,), jnp.float32)]
        + [pltpu.VMEM_SHARED((NA, DP), jnp.float32),
           pltpu.SemaphoreType.DMA]
    ),
)(_stage2_body)


def _stage3_body(acc_hbm, bias_hbm, uidx_hbm, iidx_hbm, wp_hbm, bv_hbm,
                 out_hbm, uidx_v, iidx_v, wp_v, bias_v, bv_v, idxA, idxB,
                 rowsA, rowsB, scores_v, sem):
    cid = lax.axis_index("c")
    sid = lax.axis_index("s")
    wid = cid * NS + sid
    pbase = wid * PPB
    pltpu.sync_copy(uidx_hbm.at[pl.ds(pbase, PPB)], uidx_v)
    pltpu.sync_copy(iidx_hbm.at[pl.ds(pbase, PPB)], iidx_v)
    pltpu.sync_copy(wp_hbm, wp_v)
    pltpu.sync_copy(bias_hbm, bias_v)
    pltpu.sync_copy(bv_hbm, bv_v)

    acc1 = jnp.zeros((L,), jnp.float32)
    acc2 = jnp.zeros((L,), jnp.float32)
    for j in range(D // L):
        bseg = bias_v[pl.ds(L * j, L)]
        acc1 = acc1 + bseg * wp_v[pl.ds(L * j, L)]
        acc2 = acc2 + bseg * wp_v[pl.ds(D + L * j, L)]
    bconst = jnp.sum(acc1) + jnp.sum(acc2) + bv_v[...][0]

    iota16 = lax.iota(jnp.int32, L)

    def group(g, carry):
        u16 = uidx_v[pl.ds(L * g, L)]
        i16 = iidx_v[pl.ds(L * g, L)]
        for r in range(H):
            idxA[pl.ds(L * r, L)] = u16 + r * NA
            idxB[pl.ds(L * r, L)] = i16 + r * NA
        pltpu.async_copy(acc_hbm.at[idxA], rowsA, sem).wait()
        pltpu.async_copy(acc_hbm.at[idxB], rowsB, sem).wait()
        sc16 = jnp.zeros((L,), jnp.float32)
        for p in range(L):
            def hrow(r, carry):
                g16, dn16 = carry
                row = L * r + p
                va = jnp.zeros((L,), jnp.float32)
                vb = jnp.zeros((L,), jnp.float32)
                for j in range(D // L):
                    va = va + rowsA[row, pl.ds(L * j, L)] * wp_v[pl.ds(L * j, L)]
                    vb = vb + rowsB[row, pl.ds(L * j, L)] * wp_v[pl.ds(D + L * j, L)]
                dna = rowsA[row, pl.ds(D, L)][0]
                dnb = rowsB[row, pl.ds(D, L)][0]
                g16 = jnp.where(iota16 == r, jnp.sum(va), g16)
                g16 = jnp.where(iota16 == r + H, jnp.sum(vb), g16)
                dn16 = jnp.where(iota16 == r, dna, dn16)
                dn16 = jnp.where(iota16 == r + H, dnb, dn16)
                return (g16, dn16)
            g16, dn16 = lax.fori_loop(
                0, H, hrow,
                (jnp.zeros((L,), jnp.float32), jnp.ones((L,), jnp.float32)))
            total = jnp.sum(g16 / (dn16 + 1e-16))
            sc16 = jnp.where(iota16 == p, total, sc16)
        sc16 = 1.0 / (1.0 + jnp.exp(-(sc16 * (1.0 / H) + bconst)))
        scores_v[pl.ds(L * g, L)] = sc16
        return carry
    lax.fori_loop(0, PPB // L, group, 0)
    pltpu.sync_copy(scores_v, out_hbm.at[pl.ds(pbase, PPB)])


_stage3 = functools.partial(
    pl.kernel,
    out_type=jax.ShapeDtypeStruct((B,), jnp.float32),
    mesh=plsc.VectorSubcoreMesh(core_axis_name="c", subcore_axis_name="s",
                                num_cores=NC, num_subcores=NS),
    compiler_params=pltpu.CompilerParams(needs_layout_passes=False, use_tc_tiling_on_sc=False),
    scratch_types=[
        pltpu.VMEM((PPB,), jnp.int32),
        pltpu.VMEM((PPB,), jnp.int32),
        pltpu.VMEM((2 * D,), jnp.float32),
        pltpu.VMEM((D,), jnp.float32),
        pltpu.VMEM((L,), jnp.float32),
        pltpu.VMEM((H * L,), jnp.int32),
        pltpu.VMEM((H * L,), jnp.int32),
        pltpu.VMEM((H * L, DP), jnp.float32),
        pltpu.VMEM((H * L, DP), jnp.float32),
        pltpu.VMEM((PPB,), jnp.float32),
        pltpu.SemaphoreType.DMA,
    ],
)(_stage3_body)


def kernel(edge_index, edge_type, user_indices, item_indices, entity_table,
           relation_table, W_gat, att_src, att_dst, bias_gat, W_pred, b_pred):
    del edge_type, relation_table
    pad_e = jnp.zeros((C,), jnp.int32)
    src = jnp.concatenate([edge_index[0].astype(jnp.int32), pad_e])
    dst = jnp.concatenate([edge_index[1].astype(jnp.int32), pad_e])
    xp2, s_t, t_t = _stage1(entity_table, W_gat, att_src, att_dst)
    acc = _stage2(xp2, s_t.reshape(H, N), t_t.reshape(H, N), src, dst)
    wp = W_pred.reshape(2 * D)
    bv = jnp.concatenate([b_pred.astype(jnp.float32),
                          jnp.zeros((L - 1,), jnp.float32)])
    return _stage3(acc, bias_gat, user_indices.astype(jnp.int32),
                   item_indices.astype(jnp.int32), wp, bv)


# speedup vs baseline: 19.7313x; 1.3932x over previous
import functools

import jax
import jax.numpy as jnp
from jax import lax
from jax.experimental import pallas as pl
from jax.experimental.pallas import tpu as pltpu
from jax.experimental.pallas import tpu_sc as plsc

N = 10000
D = 128
H = 8
E = 320000
B = 4096
DP = 144
PAD = DP - D

NC = 2
NS = 16
L = 16

RB = 1000
NB = N // RB

HPC = H // NC
EPT = E // NS
C = 80
CA = 48
CB = C - CA
NCHUNK = EPT // C
NA = 10240
RPT = NA // NS
ZR = 32

PPB = B // (NC * NS)


def _stage1_body(x_ref, w_ref, asrc_ref, adst_ref, xp_ref, s_ref, t_ref):
    i = pl.program_id(0)
    h = pl.program_id(1)
    xp = jnp.dot(x_ref[...], w_ref[...], preferred_element_type=jnp.float32)
    xp_ref[:, :D] = xp
    pad = (lax.broadcasted_iota(jnp.int32, (RB, PAD), 1) == 0).astype(jnp.float32)
    xp_ref[:, D:] = pad
    asrc = asrc_ref[pl.ds(h, 1), :]
    adst = adst_ref[pl.ds(h, 1), :]
    s_ref[...] = jnp.sum(xp * asrc, axis=1).reshape(1, 1, 1, RB)
    t_ref[...] = jnp.sum(xp * adst, axis=1).reshape(1, 1, 1, RB)


_stage1 = pl.pallas_call(
    _stage1_body,
    grid=(NB, H),
    in_specs=[
        pl.BlockSpec((RB, D), lambda i, h: (i, 0)),
        pl.BlockSpec((D, D), lambda i, h: (0, h)),
        pl.BlockSpec((H, D), lambda i, h: (0, 0)),
        pl.BlockSpec((H, D), lambda i, h: (0, 0)),
    ],
    out_specs=[
        pl.BlockSpec((RB, DP), lambda i, h: (h * NB + i, 0)),
        pl.BlockSpec((1, 1, 1, RB), lambda i, h: (h, i, 0, 0)),
        pl.BlockSpec((1, 1, 1, RB), lambda i, h: (h, i, 0, 0)),
    ],
    out_shape=[
        jax.ShapeDtypeStruct((H * N, DP), jnp.float32),
        jax.ShapeDtypeStruct((H, NB, 1, RB), jnp.float32),
        jax.ShapeDtypeStruct((H, NB, 1, RB), jnp.float32),
    ],
)


def _stage2_body(edges_hbm, st_hbm, tt_hbm, xp_hbm, acc_hbm,
                 s_v, t_v, rows_v,
                 ebuf0, dstA0, dstB0, idxA0, idxB0, wvA0, wvB0,
                 ebuf1, dstA1, dstB1, idxA1, idxB1, wvA1, wvB1,
                 acc_sh, semg, sema, semb):
    cid = lax.axis_index("c")
    sid = lax.axis_index("s")
    bufs0 = (ebuf0, dstA0, dstB0, idxA0, idxB0, wvA0, wvB0)
    bufs1 = (ebuf1, dstA1, dstB1, idxA1, idxB1, wvA1, wvB1)
    ebase = sid * EPT

    def compute_chunk(ci, bufs, hbase):
        ebuf, dstA, dstB, idxA, idxB, wvA, wvB = bufs
        eb = ebase + ci * C
        pltpu.sync_copy(edges_hbm.at[:, pl.ds(eb, C)], ebuf)
        for g in range(C // L):
            ina = g < CA // L
            dref, iref, wref = (dstA, idxA, wvA) if ina else (dstB, idxB, wvB)
            lofs = L * g if ina else L * g - CA
            s16 = ebuf[0, pl.ds(L * g, L)]
            d16 = ebuf[1, pl.ds(L * g, L)]
            dref[pl.ds(lofs, L)] = d16
            iref[pl.ds(lofs, L)] = s16 + hbase
            e16 = (plsc.load_gather(s_v, [s16])
                   + plsc.load_gather(t_v, [d16]))
            e16 = jnp.where(e16 >= 0, e16, 0.2 * e16)
            wref[pl.ds(lofs, L)] = jnp.exp(e16)

    def start_gA(bufs):
        pltpu.async_copy(xp_hbm.at[bufs[3]], rows_v.at[pl.ds(0, CA)], semg)

    def wait_gA(bufs):
        pltpu.make_async_copy(xp_hbm.at[bufs[3]],
                              rows_v.at[pl.ds(0, CA)], semg).wait()

    def start_gB(bufs):
        pltpu.async_copy(xp_hbm.at[bufs[4]], rows_v.at[pl.ds(CA, CB)], semg)

    def wait_gB(bufs):
        pltpu.make_async_copy(xp_hbm.at[bufs[4]],
                              rows_v.at[pl.ds(CA, CB)], semg).wait()

    def start_scA(bufs):
        pltpu.async_copy(rows_v.at[pl.ds(0, CA)], acc_sh.at[bufs[1]],
                         sema, add=True)

    def wait_scA(bufs):
        pltpu.make_async_copy(rows_v.at[pl.ds(0, CA)],
                              acc_sh.at[bufs[1]], sema).wait()

    def start_scB(bufs):
        pltpu.async_copy(rows_v.at[pl.ds(CA, CB)], acc_sh.at[bufs[2]],
                         semb, add=True)

    def wait_scB(bufs):
        pltpu.make_async_copy(rows_v.at[pl.ds(CA, CB)],
                              acc_sh.at[bufs[2]], semb).wait()

    def scale_half(wref, base, ngroups):
        def sg(g, c2):
            off = L * g
            w16 = wref[pl.ds(off, L)]
            for e in range(L):
                row = base + off + e
                for j in range(DP // L):
                    rows_v[row, pl.ds(L * j, L)] = (
                        rows_v[row, pl.ds(L * j, L)] * w16[e])
            return c2
        lax.fori_loop(0, ngroups, sg, 0)

    def do_chunk(cur, nxt, ci_next, hbase, prefetch, sb_guard):
        if prefetch:
            compute_chunk(ci_next, nxt, hbase)
        wait_gA(cur)
        if sb_guard is None:
            wait_scB(nxt)
            start_gB(cur)
        else:
            @pl.when(sb_guard)
            def _():
                wait_scB(nxt)
            start_gB(cur)
        scale_half(cur[5], 0, CA // L)
        start_scA(cur)
        wait_gB(cur)
        wait_scA(cur)
        if prefetch:
            start_gA(nxt)
        scale_half(cur[6], CA, CB // L)
        start_scB(cur)

    def zero_rows():
        def zr(i, c2):
            for j in range(DP // L):
                rows_v[i, pl.ds(L * j, L)] = jnp.zeros((L,), jnp.float32)
            return c2
        lax.fori_loop(0, C, zr, 0)

    def headbody(hh, carry):
        h = cid * HPC + hh
        hbase = h * N
        abase = h * NA
        zero_rows()
        for k in range(RPT // C):
            pltpu.sync_copy(rows_v, acc_sh.at[pl.ds(sid * RPT + k * C, C)])
        pltpu.sync_copy(st_hbm.at[h], s_v)
        pltpu.sync_copy(tt_hbm.at[h], t_v)
        plsc.subcore_barrier()

        compute_chunk(0, bufs0, hbase)
        start_gA(bufs0)

        def pairbody(k, c2):
            ci = 2 * k
            do_chunk(bufs0, bufs1, ci + 1, hbase, True, k > 0)
            do_chunk(bufs1, bufs0, ci + 2, hbase, True, None)
            return c2
        lax.fori_loop(0, NCHUNK // 2, pairbody, 0)
        wait_scB(bufs1)
        wait_gA(bufs0)

        plsc.subcore_barrier()
        pltpu.sync_copy(acc_sh.at[pl.ds(sid * RPT, RPT)],
                        acc_hbm.at[pl.ds(abase + sid * RPT, RPT)])
        plsc.subcore_barrier()
        return carry
    lax.fori_loop(0, HPC, headbody, 0)


_stage2 = functools.partial(
    pl.kernel,
    out_type=jax.ShapeDtypeStruct((H * NA, DP), jnp.float32),
    mesh=plsc.VectorSubcoreMesh(core_axis_name="c", subcore_axis_name="s",
                                num_cores=NC, num_subcores=NS),
    compiler_params=pltpu.CompilerParams(needs_layout_passes=False, use_tc_tiling_on_sc=False),
    scratch_types=(
        [pltpu.VMEM((N,), jnp.float32),
         pltpu.VMEM((N,), jnp.float32),
         pltpu.VMEM((C, DP), jnp.float32)]
        + 2 * [pltpu.VMEM((2, C), jnp.int32),
               pltpu.VMEM((CA,), jnp.int32),
               pltpu.VMEM((CB,), jnp.int32),
               pltpu.VMEM((CA,), jnp.int32),
               pltpu.VMEM((CB,), jnp.int32),
               pltpu.VMEM((CA,), jnp.float32),
               pltpu.VMEM((CB,), jnp.float32)]
        + [pltpu.VMEM_SHARED((NA, DP), jnp.float32),
           pltpu.SemaphoreType.DMA,
           pltpu.SemaphoreType.DMA,
           pltpu.SemaphoreType.DMA]
    ),
)(_stage2_body)


def _stage3_body(acc_hbm, bias_hbm, uidx_hbm, iidx_hbm, wp_hbm, bv_hbm,
                 out_hbm, uidx_v, iidx_v, wp_v, bias_v, bv_v, idxA, idxB,
                 rowsA, rowsB, scores_v, sem):
    cid = lax.axis_index("c")
    sid = lax.axis_index("s")
    wid = cid * NS + sid
    pbase = wid * PPB
    pltpu.sync_copy(uidx_hbm.at[pl.ds(pbase, PPB)], uidx_v)
    pltpu.sync_copy(iidx_hbm.at[pl.ds(pbase, PPB)], iidx_v)
    pltpu.sync_copy(wp_hbm, wp_v)
    pltpu.sync_copy(bias_hbm, bias_v)
    pltpu.sync_copy(bv_hbm, bv_v)

    acc1 = jnp.zeros((L,), jnp.float32)
    acc2 = jnp.zeros((L,), jnp.float32)
    for j in range(D // L):
        bseg = bias_v[pl.ds(L * j, L)]
        acc1 = acc1 + bseg * wp_v[pl.ds(L * j, L)]
        acc2 = acc2 + bseg * wp_v[pl.ds(D + L * j, L)]
    bconst = jnp.sum(acc1) + jnp.sum(acc2) + bv_v[...][0]

    iota16 = lax.iota(jnp.int32, L)

    def group(g, carry):
        u16 = uidx_v[pl.ds(L * g, L)]
        i16 = iidx_v[pl.ds(L * g, L)]
        for r in range(H):
            idxA[pl.ds(L * r, L)] = u16 + r * NA
            idxB[pl.ds(L * r, L)] = i16 + r * NA
        pltpu.async_copy(acc_hbm.at[idxA], rowsA, sem).wait()
        pltpu.async_copy(acc_hbm.at[idxB], rowsB, sem).wait()
        sc16 = jnp.zeros((L,), jnp.float32)
        for p in range(L):
            def hrow(r, carry):
                g16, dn16 = carry
                row = L * r + p
                va = jnp.zeros((L,), jnp.float32)
                vb = jnp.zeros((L,), jnp.float32)
                for j in range(D // L):
                    va = va + rowsA[row, pl.ds(L * j, L)] * wp_v[pl.ds(L * j, L)]
                    vb = vb + rowsB[row, pl.ds(L * j, L)] * wp_v[pl.ds(D + L * j, L)]
                dna = rowsA[row, pl.ds(D, L)][0]
                dnb = rowsB[row, pl.ds(D, L)][0]
                g16 = jnp.where(iota16 == r, jnp.sum(va), g16)
                g16 = jnp.where(iota16 == r + H, jnp.sum(vb), g16)
                dn16 = jnp.where(iota16 == r, dna, dn16)
                dn16 = jnp.where(iota16 == r + H, dnb, dn16)
                return (g16, dn16)
            g16, dn16 = lax.fori_loop(
                0, H, hrow,
                (jnp.zeros((L,), jnp.float32), jnp.ones((L,), jnp.float32)))
            total = jnp.sum(g16 / (dn16 + 1e-16))
            sc16 = jnp.where(iota16 == p, total, sc16)
        sc16 = 1.0 / (1.0 + jnp.exp(-(sc16 * (1.0 / H) + bconst)))
        scores_v[pl.ds(L * g, L)] = sc16
        return carry
    lax.fori_loop(0, PPB // L, group, 0)
    pltpu.sync_copy(scores_v, out_hbm.at[pl.ds(pbase, PPB)])


_stage3 = functools.partial(
    pl.kernel,
    out_type=jax.ShapeDtypeStruct((B,), jnp.float32),
    mesh=plsc.VectorSubcoreMesh(core_axis_name="c", subcore_axis_name="s",
                                num_cores=NC, num_subcores=NS),
    compiler_params=pltpu.CompilerParams(needs_layout_passes=False, use_tc_tiling_on_sc=False),
    scratch_types=[
        pltpu.VMEM((PPB,), jnp.int32),
        pltpu.VMEM((PPB,), jnp.int32),
        pltpu.VMEM((2 * D,), jnp.float32),
        pltpu.VMEM((D,), jnp.float32),
        pltpu.VMEM((L,), jnp.float32),
        pltpu.VMEM((H * L,), jnp.int32),
        pltpu.VMEM((H * L,), jnp.int32),
        pltpu.VMEM((H * L, DP), jnp.float32),
        pltpu.VMEM((H * L, DP), jnp.float32),
        pltpu.VMEM((PPB,), jnp.float32),
        pltpu.SemaphoreType.DMA,
    ],
)(_stage3_body)


def kernel(edge_index, edge_type, user_indices, item_indices, entity_table,
           relation_table, W_gat, att_src, att_dst, bias_gat, W_pred, b_pred):
    del edge_type, relation_table
    edges2 = jnp.concatenate(
        [edge_index.astype(jnp.int32), jnp.zeros((2, C), jnp.int32)], axis=1)
    xp2, s_t, t_t = _stage1(entity_table, W_gat, att_src, att_dst)
    acc = _stage2(edges2, s_t.reshape(H, N), t_t.reshape(H, N), xp2)
    wp = W_pred.reshape(2 * D)
    bv = jnp.concatenate([b_pred.astype(jnp.float32),
                          jnp.zeros((L - 1,), jnp.float32)])
    return _stage3(acc, bias_gat, user_indices.astype(jnp.int32),
                   item_indices.astype(jnp.int32), wp, bv)


# speedup vs baseline: 19.7446x; 1.0007x over previous
import functools

import jax
import jax.numpy as jnp
from jax import lax
from jax.experimental import pallas as pl
from jax.experimental.pallas import tpu as pltpu
from jax.experimental.pallas import tpu_sc as plsc

N = 10000
D = 128
H = 8
E = 320000
B = 4096
DP = 144
PAD = DP - D

NC = 2
NS = 16
L = 16

RB = 1000
NB = N // RB

HPC = H // NC
EPT = E // NS
C = 80
CA = 48
CB = C - CA
NCHUNK = EPT // C
NA = 10240
RPT = NA // NS
ZR = 32

PPB = B // (NC * NS)


def _stage1_body(x_ref, w_ref, asrc_ref, adst_ref, xp_ref, s_ref, t_ref):
    i = pl.program_id(0)
    h = pl.program_id(1)
    xp = jnp.dot(x_ref[...], w_ref[...], preferred_element_type=jnp.float32)
    xp_ref[:, :D] = xp
    pad = (lax.broadcasted_iota(jnp.int32, (RB, PAD), 1) == 0).astype(jnp.float32)
    xp_ref[:, D:] = pad
    asrc = asrc_ref[pl.ds(h, 1), :]
    adst = adst_ref[pl.ds(h, 1), :]
    s_ref[...] = jnp.sum(xp * asrc, axis=1).reshape(1, 1, 1, RB)
    t_ref[...] = jnp.sum(xp * adst, axis=1).reshape(1, 1, 1, RB)


_stage1 = pl.pallas_call(
    _stage1_body,
    grid=(NB, H),
    in_specs=[
        pl.BlockSpec((RB, D), lambda i, h: (i, 0)),
        pl.BlockSpec((D, D), lambda i, h: (0, h)),
        pl.BlockSpec((H, D), lambda i, h: (0, 0)),
        pl.BlockSpec((H, D), lambda i, h: (0, 0)),
    ],
    out_specs=[
        pl.BlockSpec((RB, DP), lambda i, h: (h * NB + i, 0)),
        pl.BlockSpec((1, 1, 1, RB), lambda i, h: (h, i, 0, 0)),
        pl.BlockSpec((1, 1, 1, RB), lambda i, h: (h, i, 0, 0)),
    ],
    out_shape=[
        jax.ShapeDtypeStruct((H * N, DP), jnp.float32),
        jax.ShapeDtypeStruct((H, NB, 1, RB), jnp.float32),
        jax.ShapeDtypeStruct((H, NB, 1, RB), jnp.float32),
    ],
)


def _stage2_body(edges_hbm, st_hbm, tt_hbm, xp_hbm, acc_hbm,
                 s_v, t_v, rows_v,
                 ebuf0, dstA0, dstB0, idxA0, idxB0, wvA0, wvB0,
                 ebuf1, dstA1, dstB1, idxA1, idxB1, wvA1, wvB1,
                 acc_sh, semg, sema, semb, seme):
    cid = lax.axis_index("c")
    sid = lax.axis_index("s")
    bufs0 = (ebuf0, dstA0, dstB0, idxA0, idxB0, wvA0, wvB0)
    bufs1 = (ebuf1, dstA1, dstB1, idxA1, idxB1, wvA1, wvB1)
    ebase = sid * EPT

    def start_edges(ci, bufs):
        eb = ebase + ci * C
        pltpu.async_copy(edges_hbm.at[:, pl.ds(eb, C)], bufs[0], seme)

    def wait_edges(bufs):
        pltpu.make_async_copy(edges_hbm.at[:, pl.ds(0, C)], bufs[0],
                              seme).wait()

    def compute_w(bufs, hbase):
        ebuf, dstA, dstB, idxA, idxB, wvA, wvB = bufs
        for g in range(C // L):
            ina = g < CA // L
            dref, iref, wref = (dstA, idxA, wvA) if ina else (dstB, idxB, wvB)
            lofs = L * g if ina else L * g - CA
            s16 = ebuf[0, pl.ds(L * g, L)]
            d16 = ebuf[1, pl.ds(L * g, L)]
            dref[pl.ds(lofs, L)] = d16
            iref[pl.ds(lofs, L)] = s16 + hbase
            e16 = (plsc.load_gather(s_v, [s16])
                   + plsc.load_gather(t_v, [d16]))
            e16 = jnp.where(e16 >= 0, e16, 0.2 * e16)
            wref[pl.ds(lofs, L)] = jnp.exp(e16)

    def start_gA(bufs):
        pltpu.async_copy(xp_hbm.at[bufs[3]], rows_v.at[pl.ds(0, CA)], semg)

    def wait_gA(bufs):
        pltpu.make_async_copy(xp_hbm.at[bufs[3]],
                              rows_v.at[pl.ds(0, CA)], semg).wait()

    def start_gB(bufs):
        pltpu.async_copy(xp_hbm.at[bufs[4]], rows_v.at[pl.ds(CA, CB)], semg)

    def wait_gB(bufs):
        pltpu.make_async_copy(xp_hbm.at[bufs[4]],
                              rows_v.at[pl.ds(CA, CB)], semg).wait()

    def start_scA(bufs):
        pltpu.async_copy(rows_v.at[pl.ds(0, CA)], acc_sh.at[bufs[1]],
                         sema, add=True)

    def wait_scA(bufs):
        pltpu.make_async_copy(rows_v.at[pl.ds(0, CA)],
                              acc_sh.at[bufs[1]], sema).wait()

    def start_scB(bufs):
        pltpu.async_copy(rows_v.at[pl.ds(CA, CB)], acc_sh.at[bufs[2]],
                         semb, add=True)

    def wait_scB(bufs):
        pltpu.make_async_copy(rows_v.at[pl.ds(CA, CB)],
                              acc_sh.at[bufs[2]], semb).wait()

    def scale_half(wref, base, ngroups):
        def sg(g, c2):
            off = L * g
            w16 = wref[pl.ds(off, L)]
            for e in range(L):
                row = base + off + e
                for j in range(DP // L):
                    rows_v[row, pl.ds(L * j, L)] = (
                        rows_v[row, pl.ds(L * j, L)] * w16[e])
            return c2
        lax.fori_loop(0, ngroups, sg, 0)

    def do_chunk(cur, nxt, ci_next, hbase, prefetch, sb_guard):
        if prefetch:
            wait_edges(nxt)
            start_edges(ci_next + 1, cur)
            compute_w(nxt, hbase)
        wait_gA(cur)
        if sb_guard is None:
            wait_scB(nxt)
            start_gB(cur)
        else:
            @pl.when(sb_guard)
            def _():
                wait_scB(nxt)
            start_gB(cur)
        scale_half(cur[5], 0, CA // L)
        start_scA(cur)
        wait_gB(cur)
        wait_scA(cur)
        if prefetch:
            start_gA(nxt)
        scale_half(cur[6], CA, CB // L)
        start_scB(cur)

    def zero_rows():
        def zr(i, c2):
            for j in range(DP // L):
                rows_v[i, pl.ds(L * j, L)] = jnp.zeros((L,), jnp.float32)
            return c2
        lax.fori_loop(0, C, zr, 0)

    def headbody(hh, carry):
        h = cid * HPC + hh
        hbase = h * N
        abase = h * NA
        zero_rows()
        for k in range(RPT // C):
            pltpu.sync_copy(rows_v, acc_sh.at[pl.ds(sid * RPT + k * C, C)])
        pltpu.sync_copy(st_hbm.at[h], s_v)
        pltpu.sync_copy(tt_hbm.at[h], t_v)
        plsc.subcore_barrier()

        pltpu.sync_copy(edges_hbm.at[:, pl.ds(ebase, C)], ebuf0)
        compute_w(bufs0, hbase)
        start_edges(1, bufs1)
        start_gA(bufs0)

        def pairbody(k, c2):
            ci = 2 * k
            do_chunk(bufs0, bufs1, ci + 1, hbase, True, k > 0)
            do_chunk(bufs1, bufs0, ci + 2, hbase, True, None)
            return c2
        lax.fori_loop(0, NCHUNK // 2, pairbody, 0)
        wait_edges(bufs1)
        wait_scB(bufs1)
        wait_gA(bufs0)

        plsc.subcore_barrier()
        pltpu.sync_copy(acc_sh.at[pl.ds(sid * RPT, RPT)],
                        acc_hbm.at[pl.ds(abase + sid * RPT, RPT)])
        plsc.subcore_barrier()
        return carry
    lax.fori_loop(0, HPC, headbody, 0)


_stage2 = functools.partial(
    pl.kernel,
    out_type=jax.ShapeDtypeStruct((H * NA, DP), jnp.float32),
    mesh=plsc.VectorSubcoreMesh(core_axis_name="c", subcore_axis_name="s",
                                num_cores=NC, num_subcores=NS),
    compiler_params=pltpu.CompilerParams(needs_layout_passes=False, use_tc_tiling_on_sc=False),
    scratch_types=(
        [pltpu.VMEM((N,), jnp.float32),
         pltpu.VMEM((N,), jnp.float32),
         pltpu.VMEM((C, DP), jnp.float32)]
        + 2 * [pltpu.VMEM((2, C), jnp.int32),
               pltpu.VMEM((CA,), jnp.int32),
               pltpu.VMEM((CB,), jnp.int32),
               pltpu.VMEM((CA,), jnp.int32),
               pltpu.VMEM((CB,), jnp.int32),
               pltpu.VMEM((CA,), jnp.float32),
               pltpu.VMEM((CB,), jnp.float32)]
        + [pltpu.VMEM_SHARED((NA, DP), jnp.float32),
           pltpu.SemaphoreType.DMA,
           pltpu.SemaphoreType.DMA,
           pltpu.SemaphoreType.DMA,
           pltpu.SemaphoreType.DMA]
    ),
)(_stage2_body)


def _stage3_body(acc_hbm, bias_hbm, uidx_hbm, iidx_hbm, wp_hbm, bv_hbm,
                 out_hbm, uidx_v, iidx_v, wp_v, bias_v, bv_v, idxA, idxB,
                 rowsA, rowsB, scores_v, sem):
    cid = lax.axis_index("c")
    sid = lax.axis_index("s")
    wid = cid * NS + sid
    pbase = wid * PPB
    pltpu.sync_copy(uidx_hbm.at[pl.ds(pbase, PPB)], uidx_v)
    pltpu.sync_copy(iidx_hbm.at[pl.ds(pbase, PPB)], iidx_v)
    pltpu.sync_copy(wp_hbm, wp_v)
    pltpu.sync_copy(bias_hbm, bias_v)
    pltpu.sync_copy(bv_hbm, bv_v)

    acc1 = jnp.zeros((L,), jnp.float32)
    acc2 = jnp.zeros((L,), jnp.float32)
    for j in range(D // L):
        bseg = bias_v[pl.ds(L * j, L)]
        acc1 = acc1 + bseg * wp_v[pl.ds(L * j, L)]
        acc2 = acc2 + bseg * wp_v[pl.ds(D + L * j, L)]
    bconst = jnp.sum(acc1) + jnp.sum(acc2) + bv_v[...][0]

    iota16 = lax.iota(jnp.int32, L)

    def group(g, carry):
        u16 = uidx_v[pl.ds(L * g, L)]
        i16 = iidx_v[pl.ds(L * g, L)]
        for r in range(H):
            idxA[pl.ds(L * r, L)] = u16 + r * NA
            idxB[pl.ds(L * r, L)] = i16 + r * NA
        pltpu.async_copy(acc_hbm.at[idxA], rowsA, sem).wait()
        pltpu.async_copy(acc_hbm.at[idxB], rowsB, sem).wait()
        sc16 = jnp.zeros((L,), jnp.float32)
        for p in range(L):
            def hrow(r, carry):
                g16, dn16 = carry
                row = L * r + p
                va = jnp.zeros((L,), jnp.float32)
                vb = jnp.zeros((L,), jnp.float32)
                for j in range(D // L):
                    va = va + rowsA[row, pl.ds(L * j, L)] * wp_v[pl.ds(L * j, L)]
                    vb = vb + rowsB[row, pl.ds(L * j, L)] * wp_v[pl.ds(D + L * j, L)]
                dna = rowsA[row, pl.ds(D, L)][0]
                dnb = rowsB[row, pl.ds(D, L)][0]
                g16 = jnp.where(iota16 == r, jnp.sum(va), g16)
                g16 = jnp.where(iota16 == r + H, jnp.sum(vb), g16)
                dn16 = jnp.where(iota16 == r, dna, dn16)
                dn16 = jnp.where(iota16 == r + H, dnb, dn16)
                return (g16, dn16)
            g16, dn16 = lax.fori_loop(
                0, H, hrow,
                (jnp.zeros((L,), jnp.float32), jnp.ones((L,), jnp.float32)))
            total = jnp.sum(g16 / (dn16 + 1e-16))
            sc16 = jnp.where(iota16 == p, total, sc16)
        sc16 = 1.0 / (1.0 + jnp.exp(-(sc16 * (1.0 / H) + bconst)))
        scores_v[pl.ds(L * g, L)] = sc16
        return carry
    lax.fori_loop(0, PPB // L, group, 0)
    pltpu.sync_copy(scores_v, out_hbm.at[pl.ds(pbase, PPB)])


_stage3 = functools.partial(
    pl.kernel,
    out_type=jax.ShapeDtypeStruct((B,), jnp.float32),
    mesh=plsc.VectorSubcoreMesh(core_axis_name="c", subcore_axis_name="s",
                                num_cores=NC, num_subcores=NS),
    compiler_params=pltpu.CompilerParams(needs_layout_passes=False, use_tc_tiling_on_sc=False),
    scratch_types=[
        pltpu.VMEM((PPB,), jnp.int32),
        pltpu.VMEM((PPB,), jnp.int32),
        pltpu.VMEM((2 * D,), jnp.float32),
        pltpu.VMEM((D,), jnp.float32),
        pltpu.VMEM((L,), jnp.float32),
        pltpu.VMEM((H * L,), jnp.int32),
        pltpu.VMEM((H * L,), jnp.int32),
        pltpu.VMEM((H * L, DP), jnp.float32),
        pltpu.VMEM((H * L, DP), jnp.float32),
        pltpu.VMEM((PPB,), jnp.float32),
        pltpu.SemaphoreType.DMA,
    ],
)(_stage3_body)


def kernel(edge_index, edge_type, user_indices, item_indices, entity_table,
           relation_table, W_gat, att_src, att_dst, bias_gat, W_pred, b_pred):
    del edge_type, relation_table
    edges2 = jnp.concatenate(
        [edge_index.astype(jnp.int32), jnp.zeros((2, 2 * C), jnp.int32)],
        axis=1)
    xp2, s_t, t_t = _stage1(entity_table, W_gat, att_src, att_dst)
    acc = _stage2(edges2, s_t.reshape(H, N), t_t.reshape(H, N), xp2)
    wp = W_pred.reshape(2 * D)
    bv = jnp.concatenate([b_pred.astype(jnp.float32),
                          jnp.zeros((L - 1,), jnp.float32)])
    return _stage3(acc, bias_gat, user_indices.astype(jnp.int32),
                   item_indices.astype(jnp.int32), wp, bv)


# speedup vs baseline: 22.8685x; 1.1582x over previous
import functools

import jax
import jax.numpy as jnp
from jax import lax
from jax.experimental import pallas as pl
from jax.experimental.pallas import tpu as pltpu
from jax.experimental.pallas import tpu_sc as plsc

N = 10000
D = 128
H = 8
E = 320000
B = 4096
DP = 144
DPB = 160
PAD = DPB - D
L2 = 32

NC = 2
NS = 16
L = 16

RB = 2000
NB = N // RB

HPC = H // NC
EPT = E // NS
C = 80
CA = 48
CB = C - CA
NCHUNK = EPT // C
NA = 10240
RPT = NA // NS
ZR = 32

PPB = B // (NC * NS)


def _stage1_body(x_ref, w_ref, asrc_ref, adst_ref, xp_ref, s_ref, t_ref):
    i = pl.program_id(0)
    h = pl.program_id(1)
    xp = jnp.dot(x_ref[...], w_ref[...], preferred_element_type=jnp.float32)
    xp_ref[:, :D] = xp.astype(jnp.bfloat16)
    pad = (lax.broadcasted_iota(jnp.int32, (RB, PAD), 1) == 0)
    xp_ref[:, D:] = pad.astype(jnp.bfloat16)
    asrc = asrc_ref[pl.ds(h, 1), :]
    adst = adst_ref[pl.ds(h, 1), :]
    s_ref[...] = jnp.sum(xp * asrc, axis=1).reshape(1, 1, 1, RB)
    t_ref[...] = jnp.sum(xp * adst, axis=1).reshape(1, 1, 1, RB)


_stage1 = pl.pallas_call(
    _stage1_body,
    grid=(NB, H),
    in_specs=[
        pl.BlockSpec((RB, D), lambda i, h: (i, 0)),
        pl.BlockSpec((D, D), lambda i, h: (0, h)),
        pl.BlockSpec((H, D), lambda i, h: (0, 0)),
        pl.BlockSpec((H, D), lambda i, h: (0, 0)),
    ],
    out_specs=[
        pl.BlockSpec((RB, DPB), lambda i, h: (h * NB + i, 0)),
        pl.BlockSpec((1, 1, 1, RB), lambda i, h: (h, i, 0, 0)),
        pl.BlockSpec((1, 1, 1, RB), lambda i, h: (h, i, 0, 0)),
    ],
    out_shape=[
        jax.ShapeDtypeStruct((H * N, DPB), jnp.bfloat16),
        jax.ShapeDtypeStruct((H, NB, 1, RB), jnp.float32),
        jax.ShapeDtypeStruct((H, NB, 1, RB), jnp.float32),
    ],
)


def _stage2_body(edges_hbm, st_hbm, tt_hbm, xp_hbm, acc_hbm,
                 s_v, t_v, rows_v,
                 ebuf0, dstA0, dstB0, idxA0, idxB0, wvA0, wvB0,
                 ebuf1, dstA1, dstB1, idxA1, idxB1, wvA1, wvB1,
                 acc_sh, semg, sema, semb, seme):
    cid = lax.axis_index("c")
    sid = lax.axis_index("s")
    bufs0 = (ebuf0, dstA0, dstB0, idxA0, idxB0, wvA0, wvB0)
    bufs1 = (ebuf1, dstA1, dstB1, idxA1, idxB1, wvA1, wvB1)
    ebase = sid * EPT

    def start_edges(ci, bufs):
        eb = ebase + ci * C
        pltpu.async_copy(edges_hbm.at[:, pl.ds(eb, C)], bufs[0], seme)

    def wait_edges(bufs):
        pltpu.make_async_copy(edges_hbm.at[:, pl.ds(0, C)], bufs[0],
                              seme).wait()

    def compute_w(bufs, hbase):
        ebuf, dstA, dstB, idxA, idxB, wvA, wvB = bufs
        for g in range(C // L):
            ina = g < CA // L
            dref, iref, wref = (dstA, idxA, wvA) if ina else (dstB, idxB, wvB)
            lofs = L * g if ina else L * g - CA
            s16 = ebuf[0, pl.ds(L * g, L)]
            d16 = ebuf[1, pl.ds(L * g, L)]
            dref[pl.ds(lofs, L)] = d16
            iref[pl.ds(lofs, L)] = s16 + hbase
            e16 = (plsc.load_gather(s_v, [s16])
                   + plsc.load_gather(t_v, [d16]))
            e16 = jnp.where(e16 >= 0, e16, 0.2 * e16)
            wref[pl.ds(lofs, L)] = jnp.exp(e16)

    def start_gA(bufs):
        pltpu.async_copy(xp_hbm.at[bufs[3]], rows_v.at[pl.ds(0, CA)], semg)

    def wait_gA(bufs):
        pltpu.make_async_copy(xp_hbm.at[bufs[3]],
                              rows_v.at[pl.ds(0, CA)], semg).wait()

    def start_gB(bufs):
        pltpu.async_copy(xp_hbm.at[bufs[4]], rows_v.at[pl.ds(CA, CB)], semg)

    def wait_gB(bufs):
        pltpu.make_async_copy(xp_hbm.at[bufs[4]],
                              rows_v.at[pl.ds(CA, CB)], semg).wait()

    def start_scA(bufs):
        pltpu.async_copy(rows_v.at[pl.ds(0, CA)], acc_sh.at[bufs[1]],
                         sema, add=True)

    def wait_scA(bufs):
        pltpu.make_async_copy(rows_v.at[pl.ds(0, CA)],
                              acc_sh.at[bufs[1]], sema).wait()

    def start_scB(bufs):
        pltpu.async_copy(rows_v.at[pl.ds(CA, CB)], acc_sh.at[bufs[2]],
                         semb, add=True)

    def wait_scB(bufs):
        pltpu.make_async_copy(rows_v.at[pl.ds(CA, CB)],
                              acc_sh.at[bufs[2]], semb).wait()

    def scale_half(wref, base, ngroups):
        def sg(g, c2):
            off = L * g
            w16 = wref[pl.ds(off, L)]
            for e in range(L):
                row = base + off + e
                ws = jnp.full((L,), w16[e], jnp.float32)
                wb = plsc.pack(ws, ws, format=plsc.PackFormat.INTERLEAVED)
                for j in range(DPB // L2):
                    rows_v[row, pl.ds(L2 * j, L2)] = (
                        rows_v[row, pl.ds(L2 * j, L2)] * wb)
            return c2
        lax.fori_loop(0, ngroups, sg, 0)

    def do_chunk(cur, nxt, ci_next, hbase, prefetch, sb_guard):
        if prefetch:
            wait_edges(nxt)
            start_edges(ci_next + 1, cur)
            compute_w(nxt, hbase)
        wait_gA(cur)
        if sb_guard is None:
            wait_scB(nxt)
            start_gB(cur)
        else:
            @pl.when(sb_guard)
            def _():
                wait_scB(nxt)
            start_gB(cur)
        scale_half(cur[5], 0, CA // L)
        start_scA(cur)
        wait_gB(cur)
        wait_scA(cur)
        if prefetch:
            start_gA(nxt)
        scale_half(cur[6], CA, CB // L)
        start_scB(cur)

    def zero_rows():
        def zr(i, c2):
            for j in range(DPB // L2):
                rows_v[i, pl.ds(L2 * j, L2)] = jnp.zeros((L2,), jnp.bfloat16)
            return c2
        lax.fori_loop(0, C, zr, 0)

    def headbody(hh, carry):
        h = cid * HPC + hh
        hbase = h * N
        abase = h * NA
        zero_rows()
        for k in range(RPT // C):
            pltpu.sync_copy(rows_v, acc_sh.at[pl.ds(sid * RPT + k * C, C)])
        pltpu.sync_copy(st_hbm.at[h], s_v)
        pltpu.sync_copy(tt_hbm.at[h], t_v)
        plsc.subcore_barrier()

        pltpu.sync_copy(edges_hbm.at[:, pl.ds(ebase, C)], ebuf0)
        compute_w(bufs0, hbase)
        start_edges(1, bufs1)
        start_gA(bufs0)

        def pairbody(k, c2):
            ci = 2 * k
            do_chunk(bufs0, bufs1, ci + 1, hbase, True, k > 0)
            do_chunk(bufs1, bufs0, ci + 2, hbase, True, None)
            return c2
        lax.fori_loop(0, NCHUNK // 2, pairbody, 0)
        wait_edges(bufs1)
        wait_scB(bufs1)
        wait_gA(bufs0)

        plsc.subcore_barrier()
        pltpu.sync_copy(acc_sh.at[pl.ds(sid * RPT, RPT)],
                        acc_hbm.at[pl.ds(abase + sid * RPT, RPT)])
        plsc.subcore_barrier()
        return carry
    lax.fori_loop(0, HPC, headbody, 0)


_stage2 = functools.partial(
    pl.kernel,
    out_type=jax.ShapeDtypeStruct((H * NA, DPB), jnp.bfloat16),
    mesh=plsc.VectorSubcoreMesh(core_axis_name="c", subcore_axis_name="s",
                                num_cores=NC, num_subcores=NS),
    compiler_params=pltpu.CompilerParams(needs_layout_passes=False, use_tc_tiling_on_sc=False),
    scratch_types=(
        [pltpu.VMEM((N,), jnp.float32),
         pltpu.VMEM((N,), jnp.float32),
         pltpu.VMEM((C, DPB), jnp.bfloat16)]
        + 2 * [pltpu.VMEM((2, C), jnp.int32),
               pltpu.VMEM((CA,), jnp.int32),
               pltpu.VMEM((CB,), jnp.int32),
               pltpu.VMEM((CA,), jnp.int32),
               pltpu.VMEM((CB,), jnp.int32),
               pltpu.VMEM((CA,), jnp.float32),
               pltpu.VMEM((CB,), jnp.float32)]
        + [pltpu.VMEM_SHARED((NA, DPB), jnp.bfloat16),
           pltpu.SemaphoreType.DMA,
           pltpu.SemaphoreType.DMA,
           pltpu.SemaphoreType.DMA,
           pltpu.SemaphoreType.DMA]
    ),
)(_stage2_body)


def _unpack_cols(v32):
    u = plsc.bitcast(v32, jnp.uint32)
    ev = plsc.bitcast(u << 16, jnp.float32)
    od = plsc.bitcast(u & jnp.uint32(0xFFFF0000), jnp.float32)
    return ev, od


def _stage3_body(acc_hbm, biasr_hbm, uidx_hbm, iidx_hbm, wpr_hbm, bv_hbm,
                 out_hbm, uidx_v, iidx_v, wpr_v, biasr_v, bv_v, idxA, idxB,
                 rowsA, rowsB, scores_v, sem):
    cid = lax.axis_index("c")
    sid = lax.axis_index("s")
    wid = cid * NS + sid
    pbase = wid * PPB
    pltpu.sync_copy(uidx_hbm.at[pl.ds(pbase, PPB)], uidx_v)
    pltpu.sync_copy(iidx_hbm.at[pl.ds(pbase, PPB)], iidx_v)
    pltpu.sync_copy(wpr_hbm, wpr_v)
    pltpu.sync_copy(biasr_hbm, biasr_v)
    pltpu.sync_copy(bv_hbm, bv_v)

    acc1 = jnp.zeros((L,), jnp.float32)
    acc2 = jnp.zeros((L,), jnp.float32)
    for j in range(D // L):
        bseg = biasr_v[pl.ds(L * j, L)]
        acc1 = acc1 + bseg * wpr_v[pl.ds(L * j, L)]
        acc2 = acc2 + bseg * wpr_v[pl.ds(D + L * j, L)]
    bconst = jnp.sum(acc1) + jnp.sum(acc2) + bv_v[...][0]

    iota16 = lax.iota(jnp.int32, L)

    def group(g, carry):
        u16 = uidx_v[pl.ds(L * g, L)]
        i16 = iidx_v[pl.ds(L * g, L)]
        for r in range(H):
            idxA[pl.ds(L * r, L)] = u16 + r * NA
            idxB[pl.ds(L * r, L)] = i16 + r * NA
        pltpu.async_copy(acc_hbm.at[idxA], rowsA, sem).wait()
        pltpu.async_copy(acc_hbm.at[idxB], rowsB, sem).wait()
        sc16 = jnp.zeros((L,), jnp.float32)
        for p in range(L):
            def hrow(r, carry):
                g16, dn16 = carry
                row = L * r + p
                va = jnp.zeros((L,), jnp.float32)
                vb = jnp.zeros((L,), jnp.float32)
                for j in range(D // L2):
                    eva, oda = _unpack_cols(rowsA[row, pl.ds(L2 * j, L2)])
                    evb, odb = _unpack_cols(rowsB[row, pl.ds(L2 * j, L2)])
                    va = (va + eva * wpr_v[pl.ds(L * j, L)]
                          + oda * wpr_v[pl.ds(D // 2 + L * j, L)])
                    vb = (vb + evb * wpr_v[pl.ds(D + L * j, L)]
                          + odb * wpr_v[pl.ds(D + D // 2 + L * j, L)])
                dna = _unpack_cols(rowsA[row, pl.ds(D, L2)])[0][0]
                dnb = _unpack_cols(rowsB[row, pl.ds(D, L2)])[0][0]
                g16 = jnp.where(iota16 == r, jnp.sum(va), g16)
                g16 = jnp.where(iota16 == r + H, jnp.sum(vb), g16)
                dn16 = jnp.where(iota16 == r, dna, dn16)
                dn16 = jnp.where(iota16 == r + H, dnb, dn16)
                return (g16, dn16)
            g16, dn16 = lax.fori_loop(
                0, H, hrow,
                (jnp.zeros((L,), jnp.float32), jnp.ones((L,), jnp.float32)))
            total = jnp.sum(g16 / (dn16 + 1e-16))
            sc16 = jnp.where(iota16 == p, total, sc16)
        sc16 = 1.0 / (1.0 + jnp.exp(-(sc16 * (1.0 / H) + bconst)))
        scores_v[pl.ds(L * g, L)] = sc16
        return carry
    lax.fori_loop(0, PPB // L, group, 0)
    pltpu.sync_copy(scores_v, out_hbm.at[pl.ds(pbase, PPB)])


_stage3 = functools.partial(
    pl.kernel,
    out_type=jax.ShapeDtypeStruct((B,), jnp.float32),
    mesh=plsc.VectorSubcoreMesh(core_axis_name="c", subcore_axis_name="s",
                                num_cores=NC, num_subcores=NS),
    compiler_params=pltpu.CompilerParams(needs_layout_passes=False, use_tc_tiling_on_sc=False),
    scratch_types=[
        pltpu.VMEM((PPB,), jnp.int32),
        pltpu.VMEM((PPB,), jnp.int32),
        pltpu.VMEM((2 * D,), jnp.float32),
        pltpu.VMEM((D,), jnp.float32),
        pltpu.VMEM((L,), jnp.float32),
        pltpu.VMEM((H * L,), jnp.int32),
        pltpu.VMEM((H * L,), jnp.int32),
        pltpu.VMEM((H * L, DPB), jnp.bfloat16),
        pltpu.VMEM((H * L, DPB), jnp.bfloat16),
        pltpu.VMEM((PPB,), jnp.float32),
        pltpu.SemaphoreType.DMA,
    ],
)(_stage3_body)


def kernel(edge_index, edge_type, user_indices, item_indices, entity_table,
           relation_table, W_gat, att_src, att_dst, bias_gat, W_pred, b_pred):
    del edge_type, relation_table
    edges2 = jnp.concatenate(
        [edge_index.astype(jnp.int32), jnp.zeros((2, 2 * C), jnp.int32)],
        axis=1)
    xp2, s_t, t_t = _stage1(entity_table, W_gat, att_src, att_dst)
    acc = _stage2(edges2, s_t.reshape(H, N), t_t.reshape(H, N), xp2)
    wpr = jnp.transpose(W_pred.reshape(2, D // L2, L, 2),
                        (0, 3, 1, 2)).reshape(2 * D)
    biasr = jnp.transpose(bias_gat.reshape(D // L2, L, 2),
                          (2, 0, 1)).reshape(D)
    bv = jnp.concatenate([b_pred.astype(jnp.float32),
                          jnp.zeros((L - 1,), jnp.float32)])
    return _stage3(acc, biasr, user_indices.astype(jnp.int32),
                   item_indices.astype(jnp.int32), wpr, bv)


# speedup vs baseline: 24.0241x; 1.0505x over previous
import functools

import jax
import jax.numpy as jnp
from jax import lax
from jax.experimental import pallas as pl
from jax.experimental.pallas import tpu as pltpu
from jax.experimental.pallas import tpu_sc as plsc

N = 10000
D = 128
H = 8
E = 320000
B = 4096
DP = 144
DPB = 160
PAD = DPB - D
L2 = 32

NC = 2
NS = 16
L = 16

RB = 2000
NB = N // RB

HPC = H // NC
C = 128
CA = 64
CB = C - CA
NCHUNK = 158
EPT = NCHUNK * C
TN = 10016
NA = 10240
RPT = NA // NS
ZR = 32

PPB = B // (NC * NS)


def _stage1_body(x_ref, w_ref, asrc_ref, adst_ref, xp_ref, s_ref, t_ref):
    i = pl.program_id(0)
    h = pl.program_id(1)
    xp = jnp.dot(x_ref[...], w_ref[...], preferred_element_type=jnp.float32)
    xp_ref[:, :D] = xp.astype(jnp.bfloat16)
    pad = (lax.broadcasted_iota(jnp.int32, (RB, PAD), 1) == 0)
    xp_ref[:, D:] = pad.astype(jnp.bfloat16)
    asrc = asrc_ref[pl.ds(h, 1), :]
    adst = adst_ref[pl.ds(h, 1), :]
    s_ref[...] = jnp.sum(xp * asrc, axis=1).reshape(1, 1, 1, RB)
    t_ref[...] = jnp.sum(xp * adst, axis=1).reshape(1, 1, 1, RB)


_stage1 = pl.pallas_call(
    _stage1_body,
    grid=(NB, H),
    in_specs=[
        pl.BlockSpec((RB, D), lambda i, h: (i, 0)),
        pl.BlockSpec((D, D), lambda i, h: (0, h)),
        pl.BlockSpec((H, D), lambda i, h: (0, 0)),
        pl.BlockSpec((H, D), lambda i, h: (0, 0)),
    ],
    out_specs=[
        pl.BlockSpec((RB, DPB), lambda i, h: (h * NB + i, 0)),
        pl.BlockSpec((1, 1, 1, RB), lambda i, h: (h, i, 0, 0)),
        pl.BlockSpec((1, 1, 1, RB), lambda i, h: (h, i, 0, 0)),
    ],
    out_shape=[
        jax.ShapeDtypeStruct((H * N, DPB), jnp.bfloat16),
        jax.ShapeDtypeStruct((H, NB, 1, RB), jnp.float32),
        jax.ShapeDtypeStruct((H, NB, 1, RB), jnp.float32),
    ],
)


def _stage2_body(edges_hbm, st_hbm, tt_hbm, xp_hbm, acc_hbm,
                 s_v, t_v, rows_v,
                 ebuf0, dstA0, dstB0, idxA0, idxB0, wvA0, wvB0,
                 ebuf1, dstA1, dstB1, idxA1, idxB1, wvA1, wvB1,
                 acc_sh, semg, sema, semb, seme):
    cid = lax.axis_index("c")
    sid = lax.axis_index("s")
    bufs0 = (ebuf0, dstA0, dstB0, idxA0, idxB0, wvA0, wvB0)
    bufs1 = (ebuf1, dstA1, dstB1, idxA1, idxB1, wvA1, wvB1)
    ebase = sid * EPT

    def start_edges(ci, bufs):
        eb = ebase + ci * C
        pltpu.async_copy(edges_hbm.at[:, pl.ds(eb, C)], bufs[0], seme)

    def wait_edges(bufs):
        pltpu.make_async_copy(edges_hbm.at[:, pl.ds(0, C)], bufs[0],
                              seme).wait()

    def compute_w(bufs, hbase):
        ebuf, dstA, dstB, idxA, idxB, wvA, wvB = bufs
        for g in range(C // L):
            ina = g < CA // L
            dref, iref, wref = (dstA, idxA, wvA) if ina else (dstB, idxB, wvB)
            lofs = L * g if ina else L * g - CA
            s16 = ebuf[0, pl.ds(L * g, L)]
            d16 = ebuf[1, pl.ds(L * g, L)]
            dref[pl.ds(lofs, L)] = d16
            iref[pl.ds(lofs, L)] = s16 + hbase
            e16 = (plsc.load_gather(s_v, [s16])
                   + plsc.load_gather(t_v, [d16]))
            e16 = jnp.where(e16 >= 0, e16, 0.2 * e16)
            wref[pl.ds(lofs, L)] = jnp.exp(e16)

    def start_gA(bufs):
        pltpu.async_copy(xp_hbm.at[bufs[3]], rows_v.at[pl.ds(0, CA)], semg)

    def wait_gA(bufs):
        pltpu.make_async_copy(xp_hbm.at[bufs[3]],
                              rows_v.at[pl.ds(0, CA)], semg).wait()

    def start_gB(bufs):
        pltpu.async_copy(xp_hbm.at[bufs[4]], rows_v.at[pl.ds(CA, CB)], semg)

    def wait_gB(bufs):
        pltpu.make_async_copy(xp_hbm.at[bufs[4]],
                              rows_v.at[pl.ds(CA, CB)], semg).wait()

    def start_scA(bufs):
        pltpu.async_copy(rows_v.at[pl.ds(0, CA)], acc_sh.at[bufs[1]],
                         sema, add=True)

    def wait_scA(bufs):
        pltpu.make_async_copy(rows_v.at[pl.ds(0, CA)],
                              acc_sh.at[bufs[1]], sema).wait()

    def start_scB(bufs):
        pltpu.async_copy(rows_v.at[pl.ds(CA, CB)], acc_sh.at[bufs[2]],
                         semb, add=True)

    def wait_scB(bufs):
        pltpu.make_async_copy(rows_v.at[pl.ds(CA, CB)],
                              acc_sh.at[bufs[2]], semb).wait()

    def scale_half(wref, base, ngroups):
        def sg(g, c2):
            off = L * g
            w16 = wref[pl.ds(off, L)]
            for e in range(L):
                row = base + off + e
                ws = jnp.full((L,), w16[e], jnp.float32)
                wb = plsc.pack(ws, ws, format=plsc.PackFormat.INTERLEAVED)
                for j in range(DPB // L2):
                    rows_v[row, pl.ds(L2 * j, L2)] = (
                        rows_v[row, pl.ds(L2 * j, L2)] * wb)
            return c2
        lax.fori_loop(0, ngroups, sg, 0)

    def do_chunk(cur, nxt, ci_next, hbase, prefetch, sb_guard):
        if prefetch:
            wait_edges(nxt)
            start_edges(ci_next + 1, cur)
            compute_w(nxt, hbase)
        wait_gA(cur)
        if sb_guard is None:
            wait_scB(nxt)
            start_gB(cur)
        else:
            @pl.when(sb_guard)
            def _():
                wait_scB(nxt)
            start_gB(cur)
        scale_half(cur[5], 0, CA // L)
        start_scA(cur)
        wait_gB(cur)
        wait_scA(cur)
        if prefetch:
            start_gA(nxt)
        scale_half(cur[6], CA, CB // L)
        start_scB(cur)

    def zero_rows():
        def zr(i, c2):
            for j in range(DPB // L2):
                rows_v[i, pl.ds(L2 * j, L2)] = jnp.zeros((L2,), jnp.bfloat16)
            return c2
        lax.fori_loop(0, C, zr, 0)

    def headbody(hh, carry):
        h = cid * HPC + hh
        hbase = h * N
        abase = h * NA
        zero_rows()
        for k in range(RPT // C):
            pltpu.sync_copy(rows_v, acc_sh.at[pl.ds(sid * RPT + k * C, C)])
        pltpu.sync_copy(st_hbm.at[h], s_v)
        pltpu.sync_copy(tt_hbm.at[h], t_v)
        plsc.subcore_barrier()

        pltpu.sync_copy(edges_hbm.at[:, pl.ds(ebase, C)], ebuf0)
        compute_w(bufs0, hbase)
        start_edges(1, bufs1)
        start_gA(bufs0)

        def pairbody(k, c2):
            ci = 2 * k
            do_chunk(bufs0, bufs1, ci + 1, hbase, True, k > 0)
            do_chunk(bufs1, bufs0, ci + 2, hbase, True, None)
            return c2
        lax.fori_loop(0, NCHUNK // 2, pairbody, 0)
        wait_edges(bufs1)
        wait_scB(bufs1)
        wait_gA(bufs0)

        plsc.subcore_barrier()
        pltpu.sync_copy(acc_sh.at[pl.ds(sid * RPT, RPT)],
                        acc_hbm.at[pl.ds(abase + sid * RPT, RPT)])
        plsc.subcore_barrier()
        return carry
    lax.fori_loop(0, HPC, headbody, 0)


_stage2 = functools.partial(
    pl.kernel,
    out_type=jax.ShapeDtypeStruct((H * NA, DPB), jnp.bfloat16),
    mesh=plsc.VectorSubcoreMesh(core_axis_name="c", subcore_axis_name="s",
                                num_cores=NC, num_subcores=NS),
    compiler_params=pltpu.CompilerParams(needs_layout_passes=False, use_tc_tiling_on_sc=False),
    scratch_types=(
        [pltpu.VMEM((N,), jnp.float32),
         pltpu.VMEM((TN,), jnp.float32),
         pltpu.VMEM((C, DPB), jnp.bfloat16)]
        + 2 * [pltpu.VMEM((2, C), jnp.int32),
               pltpu.VMEM((CA,), jnp.int32),
               pltpu.VMEM((CB,), jnp.int32),
               pltpu.VMEM((CA,), jnp.int32),
               pltpu.VMEM((CB,), jnp.int32),
               pltpu.VMEM((CA,), jnp.float32),
               pltpu.VMEM((CB,), jnp.float32)]
        + [pltpu.VMEM_SHARED((NA, DPB), jnp.bfloat16),
           pltpu.SemaphoreType.DMA,
           pltpu.SemaphoreType.DMA,
           pltpu.SemaphoreType.DMA,
           pltpu.SemaphoreType.DMA]
    ),
)(_stage2_body)


def _unpack_cols(v32):
    u = plsc.bitcast(v32, jnp.uint32)
    ev = plsc.bitcast(u << 16, jnp.float32)
    od = plsc.bitcast(u & jnp.uint32(0xFFFF0000), jnp.float32)
    return ev, od


def _stage3_body(acc_hbm, biasr_hbm, uidx_hbm, iidx_hbm, wpr_hbm, bv_hbm,
                 out_hbm, uidx_v, iidx_v, wpr_v, biasr_v, bv_v, idxA, idxB,
                 rowsA, rowsB, scores_v, sem):
    cid = lax.axis_index("c")
    sid = lax.axis_index("s")
    wid = cid * NS + sid
    pbase = wid * PPB
    pltpu.sync_copy(uidx_hbm.at[pl.ds(pbase, PPB)], uidx_v)
    pltpu.sync_copy(iidx_hbm.at[pl.ds(pbase, PPB)], iidx_v)
    pltpu.sync_copy(wpr_hbm, wpr_v)
    pltpu.sync_copy(biasr_hbm, biasr_v)
    pltpu.sync_copy(bv_hbm, bv_v)

    acc1 = jnp.zeros((L,), jnp.float32)
    acc2 = jnp.zeros((L,), jnp.float32)
    for j in range(D // L):
        bseg = biasr_v[pl.ds(L * j, L)]
        acc1 = acc1 + bseg * wpr_v[pl.ds(L * j, L)]
        acc2 = acc2 + bseg * wpr_v[pl.ds(D + L * j, L)]
    bconst = jnp.sum(acc1) + jnp.sum(acc2) + bv_v[...][0]

    iota16 = lax.iota(jnp.int32, L)

    def group(g, carry):
        u16 = uidx_v[pl.ds(L * g, L)]
        i16 = iidx_v[pl.ds(L * g, L)]
        for r in range(H):
            idxA[pl.ds(L * r, L)] = u16 + r * NA
            idxB[pl.ds(L * r, L)] = i16 + r * NA
        pltpu.async_copy(acc_hbm.at[idxA], rowsA, sem).wait()
        pltpu.async_copy(acc_hbm.at[idxB], rowsB, sem).wait()
        sc16 = jnp.zeros((L,), jnp.float32)
        for p in range(L):
            def hrow(r, carry):
                g16, dn16 = carry
                row = L * r + p
                va = jnp.zeros((L,), jnp.float32)
                vb = jnp.zeros((L,), jnp.float32)
                for j in range(D // L2):
                    eva, oda = _unpack_cols(rowsA[row, pl.ds(L2 * j, L2)])
                    evb, odb = _unpack_cols(rowsB[row, pl.ds(L2 * j, L2)])
                    va = (va + eva * wpr_v[pl.ds(L * j, L)]
                          + oda * wpr_v[pl.ds(D // 2 + L * j, L)])
                    vb = (vb + evb * wpr_v[pl.ds(D + L * j, L)]
                          + odb * wpr_v[pl.ds(D + D // 2 + L * j, L)])
                dna = _unpack_cols(rowsA[row, pl.ds(D, L2)])[0][0]
                dnb = _unpack_cols(rowsB[row, pl.ds(D, L2)])[0][0]
                g16 = jnp.where(iota16 == r, jnp.sum(va), g16)
                g16 = jnp.where(iota16 == r + H, jnp.sum(vb), g16)
                dn16 = jnp.where(iota16 == r, dna, dn16)
                dn16 = jnp.where(iota16 == r + H, dnb, dn16)
                return (g16, dn16)
            g16, dn16 = lax.fori_loop(
                0, H, hrow,
                (jnp.zeros((L,), jnp.float32), jnp.ones((L,), jnp.float32)))
            total = jnp.sum(g16 / (dn16 + 1e-16))
            sc16 = jnp.where(iota16 == p, total, sc16)
        sc16 = 1.0 / (1.0 + jnp.exp(-(sc16 * (1.0 / H) + bconst)))
        scores_v[pl.ds(L * g, L)] = sc16
        return carry
    lax.fori_loop(0, PPB // L, group, 0)
    pltpu.sync_copy(scores_v, out_hbm.at[pl.ds(pbase, PPB)])


_stage3 = functools.partial(
    pl.kernel,
    out_type=jax.ShapeDtypeStruct((B,), jnp.float32),
    mesh=plsc.VectorSubcoreMesh(core_axis_name="c", subcore_axis_name="s",
                                num_cores=NC, num_subcores=NS),
    compiler_params=pltpu.CompilerParams(needs_layout_passes=False, use_tc_tiling_on_sc=False),
    scratch_types=[
        pltpu.VMEM((PPB,), jnp.int32),
        pltpu.VMEM((PPB,), jnp.int32),
        pltpu.VMEM((2 * D,), jnp.float32),
        pltpu.VMEM((D,), jnp.float32),
        pltpu.VMEM((L,), jnp.float32),
        pltpu.VMEM((H * L,), jnp.int32),
        pltpu.VMEM((H * L,), jnp.int32),
        pltpu.VMEM((H * L, DPB), jnp.bfloat16),
        pltpu.VMEM((H * L, DPB), jnp.bfloat16),
        pltpu.VMEM((PPB,), jnp.float32),
        pltpu.SemaphoreType.DMA,
    ],
)(_stage3_body)


def kernel(edge_index, edge_type, user_indices, item_indices, entity_table,
           relation_table, W_gat, att_src, att_dst, bias_gat, W_pred, b_pred):
    del edge_type, relation_table
    srcs = edge_index[0].astype(jnp.int32).reshape(NS, E // NS)
    dsts = edge_index[1].astype(jnp.int32).reshape(NS, E // NS)
    padn = EPT - E // NS
    srcs = jnp.pad(srcs, ((0, 0), (0, padn)))
    dsts = jnp.pad(dsts, ((0, 0), (0, padn)), constant_values=N)
    edges2 = jnp.stack([srcs.reshape(-1), dsts.reshape(-1)])
    edges2 = jnp.pad(edges2, ((0, 0), (0, 2 * C)))
    xp2, s_t, t_t = _stage1(entity_table, W_gat, att_src, att_dst)
    t_tp = jnp.pad(t_t.reshape(H, N), ((0, 0), (0, TN - N)))
    acc = _stage2(edges2, s_t.reshape(H, N), t_tp, xp2)
    wpr = jnp.transpose(W_pred.reshape(2, D // L2, L, 2),
                        (0, 3, 1, 2)).reshape(2 * D)
    biasr = jnp.transpose(bias_gat.reshape(D // L2, L, 2),
                          (2, 0, 1)).reshape(D)
    bv = jnp.concatenate([b_pred.astype(jnp.float32),
                          jnp.zeros((L - 1,), jnp.float32)])
    return _stage3(acc, biasr, user_indices.astype(jnp.int32),
                   item_indices.astype(jnp.int32), wpr, bv)


# speedup vs baseline: 30.2527x; 1.2593x over previous
import functools

import jax
import jax.numpy as jnp
from jax import lax
from jax.experimental import pallas as pl
from jax.experimental.pallas import tpu as pltpu
from jax.experimental.pallas import tpu_sc as plsc

N = 10000
D = 128
H = 8
E = 320000
B = 4096
DP = 144
DPB = 160
PAD = DPB - D
L2 = 32

NC = 2
NS = 16
L = 16

RB = 2000
NB = N // RB

HPC = H // NC
C = 128
CA = 64
CB = C - CA
NCHUNK = 158
EPT = NCHUNK * C
TN = 10016
NA = 10240
RPT = NA // NS
ZR = 32

PPB = B // (NC * NS)


def _stage1_body(x_ref, w_ref, asrc_ref, adst_ref, xp_ref, s_ref, t_ref):
    i = pl.program_id(0)
    h = pl.program_id(1)
    xp = jnp.dot(x_ref[...], w_ref[...], preferred_element_type=jnp.float32)
    xp_ref[:, :D] = xp.astype(jnp.bfloat16)
    pad = (lax.broadcasted_iota(jnp.int32, (RB, PAD), 1) == 0)
    xp_ref[:, D:] = pad.astype(jnp.bfloat16)
    asrc = asrc_ref[pl.ds(h, 1), :]
    adst = adst_ref[pl.ds(h, 1), :]
    s_ref[...] = jnp.sum(xp * asrc, axis=1).reshape(1, 1, 1, RB)
    t_ref[...] = jnp.sum(xp * adst, axis=1).reshape(1, 1, 1, RB)


_stage1 = pl.pallas_call(
    _stage1_body,
    grid=(NB, H),
    in_specs=[
        pl.BlockSpec((RB, D), lambda i, h: (i, 0)),
        pl.BlockSpec((D, D), lambda i, h: (0, h)),
        pl.BlockSpec((H, D), lambda i, h: (0, 0)),
        pl.BlockSpec((H, D), lambda i, h: (0, 0)),
    ],
    out_specs=[
        pl.BlockSpec((RB, DPB), lambda i, h: (h * NB + i, 0)),
        pl.BlockSpec((1, 1, 1, RB), lambda i, h: (h, i, 0, 0)),
        pl.BlockSpec((1, 1, 1, RB), lambda i, h: (h, i, 0, 0)),
    ],
    out_shape=[
        jax.ShapeDtypeStruct((H * N, DPB), jnp.bfloat16),
        jax.ShapeDtypeStruct((H, NB, 1, RB), jnp.float32),
        jax.ShapeDtypeStruct((H, NB, 1, RB), jnp.float32),
    ],
)


def _stage2_body(edges_hbm, st_hbm, tt_hbm, xp_hbm, acc_hbm,
                 s_v, t_v, rows0, rows1,
                 ebuf0, dstv0, idxv0, wv0,
                 ebuf1, dstv1, idxv1, wv1,
                 acc_sh, semg, semsc, seme):
    cid = lax.axis_index("c")
    sid = lax.axis_index("s")
    bufs0 = (ebuf0, dstv0, idxv0, wv0)
    bufs1 = (ebuf1, dstv1, idxv1, wv1)
    ebase = sid * EPT

    def start_edges(ci, bufs):
        eb = ebase + ci * C
        pltpu.async_copy(edges_hbm.at[:, pl.ds(eb, C)], bufs[0], seme)

    def wait_edges(bufs):
        pltpu.make_async_copy(edges_hbm.at[:, pl.ds(0, C)], bufs[0],
                              seme).wait()

    def compute_w(bufs, hbase):
        ebuf, dstv, idxv, wv = bufs
        for g in range(C // L):
            s16 = ebuf[0, pl.ds(L * g, L)]
            d16 = ebuf[1, pl.ds(L * g, L)]
            dstv[pl.ds(L * g, L)] = d16
            idxv[pl.ds(L * g, L)] = s16 + hbase
            e16 = (plsc.load_gather(s_v, [s16])
                   + plsc.load_gather(t_v, [d16]))
            e16 = jnp.where(e16 >= 0, e16, 0.2 * e16)
            wv[pl.ds(L * g, L)] = jnp.exp(e16)

    def start_g(bufs, rows):
        pltpu.async_copy(xp_hbm.at[bufs[2]], rows, semg)

    def wait_g(bufs, rows):
        pltpu.make_async_copy(xp_hbm.at[bufs[2]], rows, semg).wait()

    def start_sc(bufs, rows):
        pltpu.async_copy(rows, acc_sh.at[bufs[1]], semsc, add=True)

    def wait_sc(bufs, rows):
        pltpu.make_async_copy(rows, acc_sh.at[bufs[1]], semsc).wait()

    def scale(bufs, rows):
        wv = bufs[3]
        def sg(g, c2):
            off = L * g
            w16 = wv[pl.ds(off, L)]
            for e in range(L):
                row = off + e
                ws = jnp.full((L,), w16[e], jnp.float32)
                wb = plsc.pack(ws, ws, format=plsc.PackFormat.INTERLEAVED)
                for j in range(DPB // L2):
                    rows[row, pl.ds(L2 * j, L2)] = (
                        rows[row, pl.ds(L2 * j, L2)] * wb)
            return c2
        lax.fori_loop(0, C // L, sg, 0)

    def do_chunk(cur, nxt, rcur, rnxt, ci_next, hbase, prefetch, sc_guard):
        if prefetch:
            wait_edges(nxt)
            start_edges(ci_next + 1, cur)
            compute_w(nxt, hbase)
        wait_g(cur, rcur)
        if sc_guard is None:
            wait_sc(nxt, rnxt)
            if prefetch:
                start_g(nxt, rnxt)
        else:
            @pl.when(sc_guard)
            def _():
                wait_sc(nxt, rnxt)
            if prefetch:
                start_g(nxt, rnxt)
        scale(cur, rcur)
        start_sc(cur, rcur)

    def zero_rows():
        def zr(i, c2):
            for j in range(DPB // L2):
                rows0[i, pl.ds(L2 * j, L2)] = jnp.zeros((L2,), jnp.bfloat16)
            return c2
        lax.fori_loop(0, C, zr, 0)

    def headbody(hh, carry):
        h = cid * HPC + hh
        hbase = h * N
        abase = h * NA
        zero_rows()
        for k in range(RPT // C):
            pltpu.sync_copy(rows0, acc_sh.at[pl.ds(sid * RPT + k * C, C)])
        pltpu.sync_copy(st_hbm.at[h], s_v)
        pltpu.sync_copy(tt_hbm.at[h], t_v)
        plsc.subcore_barrier()

        pltpu.sync_copy(edges_hbm.at[:, pl.ds(ebase, C)], ebuf0)
        compute_w(bufs0, hbase)
        start_edges(1, bufs1)
        start_g(bufs0, rows0)

        def pairbody(k, c2):
            ci = 2 * k
            do_chunk(bufs0, bufs1, rows0, rows1, ci + 1, hbase, True, k > 0)
            do_chunk(bufs1, bufs0, rows1, rows0, ci + 2, hbase, True, None)
            return c2
        lax.fori_loop(0, NCHUNK // 2, pairbody, 0)
        wait_edges(bufs1)
        wait_g(bufs0, rows0)
        wait_sc(bufs1, rows1)

        plsc.subcore_barrier()
        pltpu.sync_copy(acc_sh.at[pl.ds(sid * RPT, RPT)],
                        acc_hbm.at[pl.ds(abase + sid * RPT, RPT)])
        plsc.subcore_barrier()
        return carry
    lax.fori_loop(0, HPC, headbody, 0)


_stage2 = functools.partial(
    pl.kernel,
    out_type=jax.ShapeDtypeStruct((H * NA, DPB), jnp.bfloat16),
    mesh=plsc.VectorSubcoreMesh(core_axis_name="c", subcore_axis_name="s",
                                num_cores=NC, num_subcores=NS),
    compiler_params=pltpu.CompilerParams(needs_layout_passes=False, use_tc_tiling_on_sc=False),
    scratch_types=(
        [pltpu.VMEM((N,), jnp.float32),
         pltpu.VMEM((TN,), jnp.float32),
         pltpu.VMEM((C, DPB), jnp.bfloat16),
         pltpu.VMEM((C, DPB), jnp.bfloat16)]
        + 2 * [pltpu.VMEM((2, C), jnp.int32),
               pltpu.VMEM((C,), jnp.int32),
               pltpu.VMEM((C,), jnp.int32),
               pltpu.VMEM((C,), jnp.float32)]
        + [pltpu.VMEM_SHARED((NA, DPB), jnp.bfloat16),
           pltpu.SemaphoreType.DMA,
           pltpu.SemaphoreType.DMA,
           pltpu.SemaphoreType.DMA]
    ),
)(_stage2_body)


def _unpack_cols(v32):
    u = plsc.bitcast(v32, jnp.uint32)
    ev = plsc.bitcast(u << 16, jnp.float32)
    od = plsc.bitcast(u & jnp.uint32(0xFFFF0000), jnp.float32)
    return ev, od


def _stage3_body(acc_hbm, biasr_hbm, uidx_hbm, iidx_hbm, wpr_hbm, bv_hbm,
                 out_hbm, uidx_v, iidx_v, wpr_v, biasr_v, bv_v, idxA, idxB,
                 rowsA, rowsB, scores_v, sem):
    cid = lax.axis_index("c")
    sid = lax.axis_index("s")
    wid = cid * NS + sid
    pbase = wid * PPB
    pltpu.sync_copy(uidx_hbm.at[pl.ds(pbase, PPB)], uidx_v)
    pltpu.sync_copy(iidx_hbm.at[pl.ds(pbase, PPB)], iidx_v)
    pltpu.sync_copy(wpr_hbm, wpr_v)
    pltpu.sync_copy(biasr_hbm, biasr_v)
    pltpu.sync_copy(bv_hbm, bv_v)

    acc1 = jnp.zeros((L,), jnp.float32)
    acc2 = jnp.zeros((L,), jnp.float32)
    for j in range(D // L):
        bseg = biasr_v[pl.ds(L * j, L)]
        acc1 = acc1 + bseg * wpr_v[pl.ds(L * j, L)]
        acc2 = acc2 + bseg * wpr_v[pl.ds(D + L * j, L)]
    bconst = jnp.sum(acc1) + jnp.sum(acc2) + bv_v[...][0]

    iota16 = lax.iota(jnp.int32, L)

    def group(g, carry):
        u16 = uidx_v[pl.ds(L * g, L)]
        i16 = iidx_v[pl.ds(L * g, L)]
        for r in range(H):
            idxA[pl.ds(L * r, L)] = u16 + r * NA
            idxB[pl.ds(L * r, L)] = i16 + r * NA
        pltpu.async_copy(acc_hbm.at[idxA], rowsA, sem).wait()
        pltpu.async_copy(acc_hbm.at[idxB], rowsB, sem).wait()
        sc16 = jnp.zeros((L,), jnp.float32)
        for p in range(L):
            def hrow(r, carry):
                g16, dn16 = carry
                row = L * r + p
                va = jnp.zeros((L,), jnp.float32)
                vb = jnp.zeros((L,), jnp.float32)
                for j in range(D // L2):
                    eva, oda = _unpack_cols(rowsA[row, pl.ds(L2 * j, L2)])
                    evb, odb = _unpack_cols(rowsB[row, pl.ds(L2 * j, L2)])
                    va = (va + eva * wpr_v[pl.ds(L * j, L)]
                          + oda * wpr_v[pl.ds(D // 2 + L * j, L)])
                    vb = (vb + evb * wpr_v[pl.ds(D + L * j, L)]
                          + odb * wpr_v[pl.ds(D + D // 2 + L * j, L)])
                dna = _unpack_cols(rowsA[row, pl.ds(D, L2)])[0][0]
                dnb = _unpack_cols(rowsB[row, pl.ds(D, L2)])[0][0]
                g16 = jnp.where(iota16 == r, jnp.sum(va), g16)
                g16 = jnp.where(iota16 == r + H, jnp.sum(vb), g16)
                dn16 = jnp.where(iota16 == r, dna, dn16)
                dn16 = jnp.where(iota16 == r + H, dnb, dn16)
                return (g16, dn16)
            g16, dn16 = lax.fori_loop(
                0, H, hrow,
                (jnp.zeros((L,), jnp.float32), jnp.ones((L,), jnp.float32)))
            total = jnp.sum(g16 / (dn16 + 1e-16))
            sc16 = jnp.where(iota16 == p, total, sc16)
        sc16 = 1.0 / (1.0 + jnp.exp(-(sc16 * (1.0 / H) + bconst)))
        scores_v[pl.ds(L * g, L)] = sc16
        return carry
    lax.fori_loop(0, PPB // L, group, 0)
    pltpu.sync_copy(scores_v, out_hbm.at[pl.ds(pbase, PPB)])


_stage3 = functools.partial(
    pl.kernel,
    out_type=jax.ShapeDtypeStruct((B,), jnp.float32),
    mesh=plsc.VectorSubcoreMesh(core_axis_name="c", subcore_axis_name="s",
                                num_cores=NC, num_subcores=NS),
    compiler_params=pltpu.CompilerParams(needs_layout_passes=False, use_tc_tiling_on_sc=False),
    scratch_types=[
        pltpu.VMEM((PPB,), jnp.int32),
        pltpu.VMEM((PPB,), jnp.int32),
        pltpu.VMEM((2 * D,), jnp.float32),
        pltpu.VMEM((D,), jnp.float32),
        pltpu.VMEM((L,), jnp.float32),
        pltpu.VMEM((H * L,), jnp.int32),
        pltpu.VMEM((H * L,), jnp.int32),
        pltpu.VMEM((H * L, DPB), jnp.bfloat16),
        pltpu.VMEM((H * L, DPB), jnp.bfloat16),
        pltpu.VMEM((PPB,), jnp.float32),
        pltpu.SemaphoreType.DMA,
    ],
)(_stage3_body)


def kernel(edge_index, edge_type, user_indices, item_indices, entity_table,
           relation_table, W_gat, att_src, att_dst, bias_gat, W_pred, b_pred):
    del edge_type, relation_table
    srcs = edge_index[0].astype(jnp.int32).reshape(NS, E // NS)
    dsts = edge_index[1].astype(jnp.int32).reshape(NS, E // NS)
    padn = EPT - E // NS
    srcs = jnp.pad(srcs, ((0, 0), (0, padn)))
    dsts = jnp.pad(dsts, ((0, 0), (0, padn)), constant_values=N)
    edges2 = jnp.stack([srcs.reshape(-1), dsts.reshape(-1)])
    edges2 = jnp.pad(edges2, ((0, 0), (0, 2 * C)))
    xp2, s_t, t_t = _stage1(entity_table, W_gat, att_src, att_dst)
    t_tp = jnp.pad(t_t.reshape(H, N), ((0, 0), (0, TN - N)))
    acc = _stage2(edges2, s_t.reshape(H, N), t_tp, xp2)
    wpr = jnp.transpose(W_pred.reshape(2, D // L2, L, 2),
                        (0, 3, 1, 2)).reshape(2 * D)
    biasr = jnp.transpose(bias_gat.reshape(D // L2, L, 2),
                          (2, 0, 1)).reshape(D)
    bv = jnp.concatenate([b_pred.astype(jnp.float32),
                          jnp.zeros((L - 1,), jnp.float32)])
    return _stage3(acc, biasr, user_indices.astype(jnp.int32),
                   item_indices.astype(jnp.int32), wpr, bv)
